# baseline (speedup 1.0000x reference)
import dataclasses
import functools

import jax
import jax.numpy as jnp
from jax import lax
from jax.experimental import pallas as pl
from jax.experimental.pallas import tpu as pltpu
from jax.experimental.pallas import tpu_sc as plsc

NC, NS, L = 2, 16, 16
NW = NC * NS
EB = 128
RB = 1000

_f32 = jnp.float32


def _vsc_mesh():
    return plsc.VectorSubcoreMesh(
        core_axis_name="c", subcore_axis_name="s", num_cores=NC, num_subcores=NS
    )


def _sc_params():
    cp = pltpu.CompilerParams()
    fields = pltpu.CompilerParams.__dataclass_fields__
    if "needs_layout_passes" in fields:
        cp = dataclasses.replace(cp, needs_layout_passes=False)
    if "use_tc_tiling_on_sc" in fields:
        cp = dataclasses.replace(cp, use_tc_tiling_on_sc=False)
    return cp


def _iota16():
    return lax.iota(jnp.int32, L)


def _zero_rows(ref, nrows, width):
    nfull = width // L
    tail = width - nfull * L

    @pl.loop(0, nrows)
    def _(r):
        for q in range(nfull):
            ref[r, pl.ds(q * L, L)] = jnp.zeros((L,), _f32)
        if tail:
            rid = jnp.full((L,), r, jnp.int32)
            colt = jnp.minimum(nfull * L + _iota16(), width - 1)
            plsc.store_scatter(ref, [rid, colt], jnp.zeros((L,), _f32),
                               mask=_iota16() < tail)


def _sc_scalar_scatter(dst, edge_attr, n_nodes):
    e_total = dst.shape[0]
    nblk = e_total // EB
    w0 = 8
    rows_per = (n_nodes // NS) & ~7
    rem = n_nodes - rows_per * NS
    zr = 48

    @functools.partial(
        pl.kernel,
        out_type=jax.ShapeDtypeStruct((NC, n_nodes, w0), _f32),
        mesh=_vsc_mesh(),
        scratch_types=[
            pltpu.VMEM((EB,), jnp.int32),
            pltpu.VMEM((EB,), _f32),
            pltpu.VMEM((EB, w0), _f32),
            pltpu.VMEM((zr, w0), _f32),
            pltpu.VMEM_SHARED((n_nodes, w0), _f32),
        ],
        compiler_params=_sc_params(),
    )
    def k(dst_hbm, ea_hbm, out_hbm, di, ea_v, rows_v, zb, acc):
        cid = lax.axis_index("c")
        sid = lax.axis_index("s")
        wid = sid * NC + cid

        _zero_rows(zb, zr, w0)
        _zero_rows(rows_v, EB, w0)

        @pl.loop(0, EB // L)
        def _(g):
            rid = g * L + _iota16()
            cid1 = jnp.full((L,), 1, jnp.int32)
            plsc.store_scatter(rows_v, [rid, cid1], jnp.ones((L,), _f32))

        @pl.loop(0, rows_per, step=zr)
        def _(j):
            pltpu.sync_copy(zb, acc.at[pl.ds(sid * rows_per + j, zr)])

        @pl.when(sid == NS - 1)
        def _():
            pltpu.sync_copy(zb.at[pl.ds(0, rem)],
                            acc.at[pl.ds(NS * rows_per, rem)])

        plsc.subcore_barrier()

        @pl.loop(wid, nblk, step=NW)
        def _(b):
            base = b * EB
            pltpu.sync_copy(dst_hbm.at[pl.ds(base, EB)], di)
            pltpu.sync_copy(ea_hbm.at[pl.ds(base, EB)], ea_v)

            @pl.loop(0, EB // L)
            def _(g):
                rid = g * L + _iota16()
                cid0 = jnp.zeros((L,), jnp.int32)
                plsc.store_scatter(rows_v, [rid, cid0], ea_v[pl.ds(g * L, L)])

            pltpu.sync_copy(rows_v, acc.at[di], add=True)

        plsc.subcore_barrier()
        pltpu.sync_copy(
            acc.at[pl.ds(sid * rows_per, rows_per)],
            out_hbm.at[cid, pl.ds(sid * rows_per, rows_per)],
        )

        @pl.when(sid == NS - 1)
        def _():
            pltpu.sync_copy(
                acc.at[pl.ds(NS * rows_per, rem)],
                out_hbm.at[cid, pl.ds(NS * rows_per, rem)],
            )

    return k(dst, edge_attr)


def _sc_edge_pass(xl, xr, src, dst, attf, heads):
    n_nodes, d = xl.shape
    h_ = heads
    c_ = d // h_
    w = d + h_
    nblk = src.shape[0]
    nb_main = nblk // NW
    tail = nblk - nb_main * NW
    rows_per = (n_nodes // NS) & ~7
    rem = n_nodes - rows_per * NS
    zr = 48

    @functools.partial(
        pl.kernel,
        out_type=jax.ShapeDtypeStruct((NC, n_nodes, w), _f32),
        mesh=_vsc_mesh(),
        scratch_types=[
            pltpu.VMEM((nb_main + 1, EB), jnp.int32),
            pltpu.VMEM((nb_main + 1, EB), jnp.int32),
            pltpu.VMEM((EB, d), _f32),
            pltpu.VMEM((EB, d), _f32),
            pltpu.VMEM((EB, d), _f32),
            pltpu.VMEM((EB, d), _f32),
            pltpu.VMEM((EB, w), _f32),
            pltpu.VMEM((EB, w), _f32),
            pltpu.VMEM((d,), _f32),
            pltpu.VMEM((zr, w), _f32),
            pltpu.VMEM_SHARED((n_nodes, w), _f32),
            pltpu.SemaphoreType.DMA,
            pltpu.SemaphoreType.DMA,
            pltpu.SemaphoreType.DMA,
            pltpu.SemaphoreType.DMA,
        ],
        compiler_params=_sc_params(),
    )
    def k(xl_hbm, xr_hbm, src_hbm, dst_hbm, att_hbm, out_hbm,
          silo, dilo, xlb0, xlb1, xrb0, xrb1, ob0, ob1, att_v, zb, acc,
          gsem0, gsem1, ssem0, ssem1):
        cid = lax.axis_index("c")
        sid = lax.axis_index("s")
        wid = sid * NC + cid
        bufs = ((xlb0, xrb0, ob0, gsem0, ssem0),
                (xlb1, xrb1, ob1, gsem1, ssem1))

        pltpu.sync_copy(att_hbm, att_v)
        _zero_rows(zb, zr, w)

        pltpu.sync_copy(src_hbm.at[pl.ds(wid * nb_main, nb_main)],
                        silo.at[pl.ds(0, nb_main)])
        pltpu.sync_copy(dst_hbm.at[pl.ds(wid * nb_main, nb_main)],
                        dilo.at[pl.ds(0, nb_main)])
        if tail:
            @pl.when(wid < tail)
            def _():
                pltpu.sync_copy(src_hbm.at[NW * nb_main + wid],
                                silo.at[nb_main])
                pltpu.sync_copy(dst_hbm.at[NW * nb_main + wid],
                                dilo.at[nb_main])

        @pl.loop(0, rows_per, step=zr)
        def _(j):
            pltpu.sync_copy(zb, acc.at[pl.ds(sid * rows_per + j, zr)])

        @pl.when(sid == NS - 1)
        def _():
            pltpu.sync_copy(zb.at[pl.ds(0, rem)],
                            acc.at[pl.ds(NS * rows_per, rem)])

        plsc.subcore_barrier()

        def issue_gather(kb, p):
            xlb_, xrb_, _, gsem_, _ = bufs[p]
            pltpu.async_copy(xl_hbm.at[silo.at[kb]], xlb_, gsem_)
            pltpu.async_copy(xr_hbm.at[dilo.at[kb]], xrb_, gsem_)

        def wait_gather(kb, p):
            xlb_, xrb_, _, gsem_, _ = bufs[p]
            pltpu.make_async_copy(xl_hbm.at[silo.at[kb]], xlb_, gsem_).wait()
            pltpu.make_async_copy(xr_hbm.at[dilo.at[kb]], xrb_, gsem_).wait()

        def issue_scatter(kb, p):
            _, _, ob_, _, ssem_ = bufs[p]
            pltpu.async_copy(ob_, acc.at[dilo.at[kb]], ssem_, add=True)

        def wait_scatter(kb, p):
            _, _, ob_, _, ssem_ = bufs[p]
            pltpu.make_async_copy(ob_, acc.at[dilo.at[kb]], ssem_).wait()

        def compute(p):
            xlb_, xrb_, ob_, _, _ = bufs[p]

            @plsc.parallel_loop(0, EB // L)
            def _(g):
                rid = g * L + _iota16()
                for h in range(h_):
                    accs = [jnp.zeros((L,), _f32) for _ in range(4)]
                    for cc in range(c_):
                        c = h * c_ + cc
                        colc = jnp.full((L,), c, jnp.int32)
                        xlc = plsc.load_gather(xlb_, [rid, colc])
                        xrc = plsc.load_gather(xrb_, [rid, colc])
                        av = plsc.load_gather(att_v, [colc])
                        m = xlc + xrc
                        m = jnp.maximum(m, 0.2 * m)
                        accs[cc % 4] = accs[cc % 4] + m * av
                    ex = jnp.exp((accs[0] + accs[1]) + (accs[2] + accs[3]))
                    plsc.store_scatter(
                        ob_, [rid, jnp.full((L,), d + h, jnp.int32)], ex)
                    for cc in range(c_):
                        c = h * c_ + cc
                        colc = jnp.full((L,), c, jnp.int32)
                        xlc = plsc.load_gather(xlb_, [rid, colc])
                        plsc.store_scatter(ob_, [rid, colc], xlc * ex)

        issue_gather(0, 0)

        @pl.loop(0, nb_main // 2)
        def _(j):
            for p in (0, 1):
                kb = 2 * j + p
                if p == 0:
                    issue_gather(kb + 1, 1)
                else:
                    @pl.when(kb + 1 < nb_main)
                    def _():
                        issue_gather(kb + 1, 0)
                wait_gather(kb, p)

                @pl.when(kb >= 2)
                def _():
                    wait_scatter(kb - 2, p)

                compute(p)
                issue_scatter(kb, p)

        wait_scatter(nb_main - 2, 0)
        wait_scatter(nb_main - 1, 1)

        if tail:
            @pl.when(wid < tail)
            def _():
                issue_gather(nb_main, 0)
                wait_gather(nb_main, 0)
                compute(0)
                pltpu.sync_copy(ob0, acc.at[dilo.at[nb_main]], add=True)

        plsc.subcore_barrier()
        pltpu.sync_copy(
            acc.at[pl.ds(sid * rows_per, rows_per)],
            out_hbm.at[cid, pl.ds(sid * rows_per, rows_per)],
        )

        @pl.when(sid == NS - 1)
        def _():
            pltpu.sync_copy(
                acc.at[pl.ds(NS * rows_per, rem)],
                out_hbm.at[cid, pl.ds(NS * rows_per, rem)],
            )

    return k(xl, xr, src, dst, attf)


def _dg(a, b):
    return lax.dot_general(a, b, (((1,), (1,)), ((), ())),
                           preferred_element_type=_f32)


def _tc_pre(x, acc0, we_row, be_row, wlx, wlm, bl, wrx, wrm, br):
    n, f_in = x.shape
    nb = n // RB
    hid = we_row.shape[1]
    d_out = bl.shape[1]

    def body(x_ref, a_ref, we_ref, be_ref, wlx_ref, wlm_ref, bl_ref,
             wrx_ref, wrm_ref, br_ref, xl_ref, xr_ref):
        s = a_ref[0, :, 0:1] + a_ref[1, :, 0:1]
        c = a_ref[0, :, 1:2] + a_ref[1, :, 1:2]
        msg = s * we_ref[...] + c * be_ref[...]
        xv = x_ref[...]
        xl_ref[...] = _dg(xv, wlx_ref[...]) + _dg(msg, wlm_ref[...]) + bl_ref[...]
        xr_ref[...] = _dg(xv, wrx_ref[...]) + _dg(msg, wrm_ref[...]) + br_ref[...]

    full = lambda shp: pl.BlockSpec(shp, lambda i: (0,) * len(shp))
    return pl.pallas_call(
        body,
        grid=(nb,),
        in_specs=[
            pl.BlockSpec((RB, f_in), lambda i: (i, 0)),
            pl.BlockSpec((NC, RB, acc0.shape[2]), lambda i: (0, i, 0)),
            full((1, hid)), full((1, hid)),
            full((d_out, f_in)), full((d_out, hid)), full((1, d_out)),
            full((d_out, f_in)), full((d_out, hid)), full((1, d_out)),
        ],
        out_specs=[
            pl.BlockSpec((RB, d_out), lambda i: (i, 0)),
            pl.BlockSpec((RB, d_out), lambda i: (i, 0)),
        ],
        out_shape=[
            jax.ShapeDtypeStruct((n, d_out), _f32),
            jax.ShapeDtypeStruct((n, d_out), _f32),
        ],
    )(x, acc0, we_row, be_row, wlx, wlm, bl, wrx, wrm, br)


def _tc_mid(acc1a, acc1b, xl1, xr1, att_row, bias_row, wl2, bl2, wr2, br2):
    n, d = xl1.shape
    h_ = 4
    c_ = d // h_
    wh = d // 2 + 2
    nb = n // RB
    d2 = wl2.shape[0]

    def body(a0_ref, a1_ref, xl_ref, xr_ref, att_ref, bias_ref, wl2_ref,
             bl2_ref, wr2_ref, br2_ref, xl2_ref, xr2_ref):
        dh = d // 2
        num = jnp.concatenate(
            [a0_ref[0, :, 0:dh] + a0_ref[1, :, 0:dh],
             a1_ref[0, :, 0:dh] + a1_ref[1, :, 0:dh]], axis=1)
        den = jnp.concatenate(
            [a0_ref[0, :, dh:dh + 2] + a0_ref[1, :, dh:dh + 2],
             a1_ref[0, :, dh:dh + 2] + a1_ref[1, :, dh:dh + 2]], axis=1)
        xlv = xl_ref[...]
        m = xlv + xr_ref[...]
        m = jnp.maximum(m, 0.2 * m)
        t = m * att_ref[...]
        ci = lax.broadcasted_iota(jnp.int32, (d, h_), 0) // c_
        hi = lax.broadcasted_iota(jnp.int32, (d, h_), 1)
        sel = (ci == hi).astype(_f32)
        als = lax.dot_general(t, sel, (((1,), (0,)), ((), ())),
                              preferred_element_type=_f32)
        exs = jnp.exp(als)
        den = den + exs
        ex128 = _dg(exs, sel)
        den128 = _dg(den, sel)
        out = (num + ex128 * xlv) / (den128 + 1e-16) + bias_ref[...]
        h1 = jnp.where(out > 0, out, jnp.exp(out) - 1.0)
        xl2_ref[...] = _dg(h1, wl2_ref[...]) + bl2_ref[...]
        xr2_ref[...] = _dg(h1, wr2_ref[...]) + br2_ref[...]

    full = lambda shp: pl.BlockSpec(shp, lambda i: (0,) * len(shp))
    return pl.pallas_call(
        body,
        grid=(nb,),
        in_specs=[
            pl.BlockSpec((NC, RB, wh), lambda i: (0, i, 0)),
            pl.BlockSpec((NC, RB, wh), lambda i: (0, i, 0)),
            pl.BlockSpec((RB, d), lambda i: (i, 0)),
            pl.BlockSpec((RB, d), lambda i: (i, 0)),
            full((1, d)), full((1, d)),
            full((d2, d)), full((1, d2)),
            full((d2, d)), full((1, d2)),
        ],
        out_specs=[
            pl.BlockSpec((RB, d2), lambda i: (i, 0)),
            pl.BlockSpec((RB, d2), lambda i: (i, 0)),
        ],
        out_shape=[
            jax.ShapeDtypeStruct((n, d2), _f32),
            jax.ShapeDtypeStruct((n, d2), _f32),
        ],
    )(acc1a, acc1b, xl1, xr1, att_row, bias_row, wl2, bl2, wr2, br2)


def _tc_post(acc2, xl2, xr2, att_row, bias_row, batchf, wf, bf, num_graphs):
    n, d = xl2.shape
    w = d + 1
    nb = n // RB
    ncls = wf.shape[0]

    def body(a_ref, xl_ref, xr_ref, att_ref, bias_ref, b_ref, wf_ref, bf_ref,
             out_ref, sums_ref, cnts_ref):
        i = pl.program_id(0)

        @pl.when(i == 0)
        def _():
            sums_ref[...] = jnp.zeros_like(sums_ref)
            cnts_ref[...] = jnp.zeros_like(cnts_ref)

        num = a_ref[0, :, 0:d] + a_ref[1, :, 0:d]
        den = a_ref[0, :, d:d + 1] + a_ref[1, :, d:d + 1]
        xlv = xl_ref[...]
        m = xlv + xr_ref[...]
        m = jnp.maximum(m, 0.2 * m)
        t = m * att_ref[...]
        al = jnp.sum(t, axis=1, keepdims=True)
        exs = jnp.exp(al)
        out = (num + exs * xlv) / (den + exs + 1e-16) + bias_ref[...]
        h2 = jnp.where(out > 0, out, jnp.exp(out) - 1.0)
        gi = lax.broadcasted_iota(jnp.int32, (RB, num_graphs), 1).astype(_f32)
        on = (b_ref[...] == gi).astype(_f32)
        sums_ref[...] += lax.dot_general(on, h2, (((0,), (0,)), ((), ())),
                                         preferred_element_type=_f32)
        cnts_ref[...] += lax.dot_general(on, jnp.ones_like(h2),
                                         (((0,), (0,)), ((), ())),
                                         preferred_element_type=_f32)

        @pl.when(i == nb - 1)
        def _():
            pooled = sums_ref[...] / jnp.maximum(cnts_ref[...], 1.0)
            logits = _dg(pooled, wf_ref[...]) + bf_ref[...]
            mx = jnp.max(logits, axis=1, keepdims=True)
            lse = mx + jnp.log(jnp.sum(jnp.exp(logits - mx), axis=1,
                                       keepdims=True))
            out_ref[...] = logits - lse

    full = lambda shp: pl.BlockSpec(shp, lambda i: (0,) * len(shp))
    return pl.pallas_call(
        body,
        grid=(nb,),
        in_specs=[
            pl.BlockSpec((NC, RB, w), lambda i: (0, i, 0)),
            pl.BlockSpec((RB, d), lambda i: (i, 0)),
            pl.BlockSpec((RB, d), lambda i: (i, 0)),
            full((1, d)), full((1, d)),
            pl.BlockSpec((RB, 1), lambda i: (i, 0)),
            full((ncls, d)), full((1, ncls)),
        ],
        out_specs=pl.BlockSpec((num_graphs, ncls), lambda i: (0, 0)),
        out_shape=jax.ShapeDtypeStruct((num_graphs, ncls), _f32),
        scratch_shapes=[
            pltpu.VMEM((num_graphs, d), _f32),
            pltpu.VMEM((num_graphs, d), _f32),
        ],
    )(acc2, xl2, xr2, att_row, bias_row, batchf, wf, bf)


def kernel(x, edge_index, edge_attr, batch, We, be, Wl1, bl1, Wr1, br1, att1,
           bias1, Wl2, bl2, Wr2, br2, att2, bias2, Wf, bf):
    n, f_in = x.shape
    src = edge_index[0]
    dst = edge_index[1]
    hid = We.shape[0]
    num_graphs = 64

    acc0 = _sc_scalar_scatter(dst, edge_attr, n)
    xl1, xr1 = _tc_pre(
        x, acc0,
        We[:, 0].reshape(1, hid), be.reshape(1, hid),
        Wl1[:, :f_in], Wl1[:, f_in:], bl1.reshape(1, -1),
        Wr1[:, :f_in], Wr1[:, f_in:], br1.reshape(1, -1),
    )
    src2d = src.reshape(-1, EB)
    dst2d = dst.reshape(-1, EB)
    att1f = att1.reshape(-1)
    dh = att1f.shape[0] // 2
    acc1a = _sc_edge_pass(xl1[:, :dh], xr1[:, :dh], src2d, dst2d,
                          att1f[:dh], heads=2)
    acc1b = _sc_edge_pass(xl1[:, dh:], xr1[:, dh:], src2d, dst2d,
                          att1f[dh:], heads=2)
    xl2, xr2 = _tc_mid(
        acc1a, acc1b, xl1, xr1, att1.reshape(1, -1), bias1.reshape(1, -1),
        Wl2, bl2.reshape(1, -1), Wr2, br2.reshape(1, -1),
    )
    acc2 = _sc_edge_pass(xl2, xr2, src2d, dst2d, att2.reshape(-1), heads=1)
    return _tc_post(
        acc2, xl2, xr2, att2.reshape(1, -1), bias2.reshape(1, -1),
        batch.astype(_f32).reshape(-1, 1), Wf, bf.reshape(1, -1), num_graphs,
    )


# speedup vs baseline: 1.1863x; 1.1863x over previous
import dataclasses
import functools

import jax
import jax.numpy as jnp
from jax import lax
from jax.experimental import pallas as pl
from jax.experimental.pallas import tpu as pltpu
from jax.experimental.pallas import tpu_sc as plsc

NC, NS, L = 2, 16, 16
NW = NC * NS
EB = 128
RB = 1000

_f32 = jnp.float32


def _vsc_mesh():
    return plsc.VectorSubcoreMesh(
        core_axis_name="c", subcore_axis_name="s", num_cores=NC, num_subcores=NS
    )


def _sc_params():
    cp = pltpu.CompilerParams()
    fields = pltpu.CompilerParams.__dataclass_fields__
    if "needs_layout_passes" in fields:
        cp = dataclasses.replace(cp, needs_layout_passes=False)
    if "use_tc_tiling_on_sc" in fields:
        cp = dataclasses.replace(cp, use_tc_tiling_on_sc=False)
    return cp


def _iota16():
    return lax.iota(jnp.int32, L)


def _zero_rows(ref, nrows, width):
    nfull = width // L
    tail = width - nfull * L

    @pl.loop(0, nrows)
    def _(r):
        for q in range(nfull):
            ref[r, pl.ds(q * L, L)] = jnp.zeros((L,), _f32)
        if tail:
            rid = jnp.full((L,), r, jnp.int32)
            colt = jnp.minimum(nfull * L + _iota16(), width - 1)
            plsc.store_scatter(ref, [rid, colt], jnp.zeros((L,), _f32),
                               mask=_iota16() < tail)


def _sc_scalar_scatter(dst, edge_attr, n_nodes):
    e_total = dst.shape[0]
    nblk = e_total // EB
    w0 = 8
    rows_per = (n_nodes // NS) & ~7
    rem = n_nodes - rows_per * NS
    zr = 48

    @functools.partial(
        pl.kernel,
        out_type=jax.ShapeDtypeStruct((NC, n_nodes, w0), _f32),
        mesh=_vsc_mesh(),
        scratch_types=[
            pltpu.VMEM((EB,), jnp.int32),
            pltpu.VMEM((EB,), _f32),
            pltpu.VMEM((EB, w0), _f32),
            pltpu.VMEM((zr, w0), _f32),
            pltpu.VMEM_SHARED((n_nodes, w0), _f32),
        ],
        compiler_params=_sc_params(),
    )
    def k(dst_hbm, ea_hbm, out_hbm, di, ea_v, rows_v, zb, acc):
        cid = lax.axis_index("c")
        sid = lax.axis_index("s")
        wid = sid * NC + cid

        _zero_rows(zb, zr, w0)
        _zero_rows(rows_v, EB, w0)

        @pl.loop(0, EB // L)
        def _(g):
            rid = g * L + _iota16()
            cid1 = jnp.full((L,), 1, jnp.int32)
            plsc.store_scatter(rows_v, [rid, cid1], jnp.ones((L,), _f32))

        @pl.loop(0, rows_per, step=zr)
        def _(j):
            pltpu.sync_copy(zb, acc.at[pl.ds(sid * rows_per + j, zr)])

        @pl.when(sid == NS - 1)
        def _():
            pltpu.sync_copy(zb.at[pl.ds(0, rem)],
                            acc.at[pl.ds(NS * rows_per, rem)])

        plsc.subcore_barrier()

        @pl.loop(wid, nblk, step=NW)
        def _(b):
            base = b * EB
            pltpu.sync_copy(dst_hbm.at[pl.ds(base, EB)], di)
            pltpu.sync_copy(ea_hbm.at[pl.ds(base, EB)], ea_v)

            @pl.loop(0, EB // L)
            def _(g):
                rid = g * L + _iota16()
                cid0 = jnp.zeros((L,), jnp.int32)
                plsc.store_scatter(rows_v, [rid, cid0], ea_v[pl.ds(g * L, L)])

            pltpu.sync_copy(rows_v, acc.at[di], add=True)

        plsc.subcore_barrier()
        pltpu.sync_copy(
            acc.at[pl.ds(sid * rows_per, rows_per)],
            out_hbm.at[cid, pl.ds(sid * rows_per, rows_per)],
        )

        @pl.when(sid == NS - 1)
        def _():
            pltpu.sync_copy(
                acc.at[pl.ds(NS * rows_per, rem)],
                out_hbm.at[cid, pl.ds(NS * rows_per, rem)],
            )

    return k(dst, edge_attr)


def _sc_edge_pass(xl, xr, src, dst, attf, heads):
    n_nodes, d = xl.shape
    h_ = heads
    c_ = d // h_
    w = d + h_
    nblk = src.shape[0]
    nb_main = nblk // NW
    tail = nblk - nb_main * NW
    rows_per = (n_nodes // NS) & ~7
    rem = n_nodes - rows_per * NS
    zr = 48

    @functools.partial(
        pl.kernel,
        out_type=jax.ShapeDtypeStruct((NC, n_nodes, w), _f32),
        mesh=_vsc_mesh(),
        scratch_types=[
            pltpu.VMEM((nb_main + 1, EB), jnp.int32),
            pltpu.VMEM((nb_main + 1, EB), jnp.int32),
            pltpu.VMEM((EB, d), _f32),
            pltpu.VMEM((EB, d), _f32),
            pltpu.VMEM((EB, d), _f32),
            pltpu.VMEM((EB, d), _f32),
            pltpu.VMEM((EB, w), _f32),
            pltpu.VMEM((EB, w), _f32),
            pltpu.VMEM((d,), _f32),
            pltpu.VMEM((zr, w), _f32),
            pltpu.VMEM_SHARED((n_nodes, w), _f32),
            pltpu.SemaphoreType.DMA,
            pltpu.SemaphoreType.DMA,
            pltpu.SemaphoreType.DMA,
            pltpu.SemaphoreType.DMA,
        ],
        compiler_params=_sc_params(),
    )
    def k(xl_hbm, xr_hbm, src_hbm, dst_hbm, att_hbm, out_hbm,
          silo, dilo, xlb0, xlb1, xrb0, xrb1, ob0, ob1, att_v, zb, acc,
          gsem0, gsem1, ssem0, ssem1):
        cid = lax.axis_index("c")
        sid = lax.axis_index("s")
        wid = sid * NC + cid
        bufs = ((xlb0, xrb0, ob0, gsem0, ssem0),
                (xlb1, xrb1, ob1, gsem1, ssem1))

        pltpu.sync_copy(att_hbm, att_v)
        _zero_rows(zb, zr, w)

        pltpu.sync_copy(src_hbm.at[pl.ds(wid * nb_main, nb_main)],
                        silo.at[pl.ds(0, nb_main)])
        pltpu.sync_copy(dst_hbm.at[pl.ds(wid * nb_main, nb_main)],
                        dilo.at[pl.ds(0, nb_main)])
        if tail:
            @pl.when(wid < tail)
            def _():
                pltpu.sync_copy(src_hbm.at[NW * nb_main + wid],
                                silo.at[nb_main])
                pltpu.sync_copy(dst_hbm.at[NW * nb_main + wid],
                                dilo.at[nb_main])

        @pl.loop(0, rows_per, step=zr)
        def _(j):
            pltpu.sync_copy(zb, acc.at[pl.ds(sid * rows_per + j, zr)])

        @pl.when(sid == NS - 1)
        def _():
            pltpu.sync_copy(zb.at[pl.ds(0, rem)],
                            acc.at[pl.ds(NS * rows_per, rem)])

        plsc.subcore_barrier()

        def issue_gather(kb, p):
            xlb_, xrb_, _, gsem_, _ = bufs[p]
            pltpu.async_copy(xl_hbm.at[silo.at[kb]], xlb_, gsem_)
            pltpu.async_copy(xr_hbm.at[dilo.at[kb]], xrb_, gsem_)

        def wait_gather(kb, p):
            xlb_, xrb_, _, gsem_, _ = bufs[p]
            pltpu.make_async_copy(xl_hbm.at[silo.at[kb]], xlb_, gsem_).wait()
            pltpu.make_async_copy(xr_hbm.at[dilo.at[kb]], xrb_, gsem_).wait()

        def issue_scatter(kb, p):
            _, _, ob_, _, ssem_ = bufs[p]
            pltpu.async_copy(ob_, acc.at[dilo.at[kb]], ssem_, add=True)

        def wait_scatter(kb, p):
            _, _, ob_, _, ssem_ = bufs[p]
            pltpu.make_async_copy(ob_, acc.at[dilo.at[kb]], ssem_).wait()

        def compute(p):
            xlb_, xrb_, ob_, _, _ = bufs[p]

            @pl.loop(0, EB // L)
            def _(g):
                rid = g * L + _iota16()
                for h in range(h_):
                    def a_body(cc, acc_, h=h):
                        colc = jnp.full((L,), h * c_, jnp.int32) + cc
                        xlc = plsc.load_gather(xlb_, [rid, colc])
                        xrc = plsc.load_gather(xrb_, [rid, colc])
                        av = plsc.load_gather(att_v, [colc])
                        m = xlc + xrc
                        m = jnp.maximum(m, 0.2 * m)
                        return acc_ + m * av

                    alpha = plsc.parallel_loop(
                        0, c_, unroll=4,
                        carry=jnp.zeros((L,), _f32))(a_body)
                    ex = jnp.exp(alpha)
                    plsc.store_scatter(
                        ob_, [rid, jnp.full((L,), d + h, jnp.int32)], ex)

                    def w_body(cc, h=h, ex=ex):
                        colc = jnp.full((L,), h * c_, jnp.int32) + cc
                        xlc = plsc.load_gather(xlb_, [rid, colc])
                        plsc.store_scatter(ob_, [rid, colc], xlc * ex)

                    plsc.parallel_loop(0, c_, unroll=4)(w_body)

        issue_gather(0, 0)

        @pl.loop(0, nb_main // 2)
        def _(j):
            for p in (0, 1):
                kb = 2 * j + p
                if p == 0:
                    issue_gather(kb + 1, 1)
                else:
                    @pl.when(kb + 1 < nb_main)
                    def _():
                        issue_gather(kb + 1, 0)
                wait_gather(kb, p)

                @pl.when(kb >= 2)
                def _():
                    wait_scatter(kb - 2, p)

                compute(p)
                issue_scatter(kb, p)

        wait_scatter(nb_main - 2, 0)
        wait_scatter(nb_main - 1, 1)

        if tail:
            @pl.when(wid < tail)
            def _():
                issue_gather(nb_main, 0)
                wait_gather(nb_main, 0)
                compute(0)
                pltpu.sync_copy(ob0, acc.at[dilo.at[nb_main]], add=True)

        plsc.subcore_barrier()
        pltpu.sync_copy(
            acc.at[pl.ds(sid * rows_per, rows_per)],
            out_hbm.at[cid, pl.ds(sid * rows_per, rows_per)],
        )

        @pl.when(sid == NS - 1)
        def _():
            pltpu.sync_copy(
                acc.at[pl.ds(NS * rows_per, rem)],
                out_hbm.at[cid, pl.ds(NS * rows_per, rem)],
            )

    return k(xl, xr, src, dst, attf)


def _dg(a, b):
    return lax.dot_general(a, b, (((1,), (1,)), ((), ())),
                           preferred_element_type=_f32)


def _tc_pre(x, acc0, we_row, be_row, wlx, wlm, bl, wrx, wrm, br):
    n, f_in = x.shape
    nb = n // RB
    hid = we_row.shape[1]
    d_out = bl.shape[1]

    def body(x_ref, a_ref, we_ref, be_ref, wlx_ref, wlm_ref, bl_ref,
             wrx_ref, wrm_ref, br_ref, xl_ref, xr_ref):
        s = a_ref[0, :, 0:1] + a_ref[1, :, 0:1]
        c = a_ref[0, :, 1:2] + a_ref[1, :, 1:2]
        msg = s * we_ref[...] + c * be_ref[...]
        xv = x_ref[...]
        xl_ref[...] = _dg(xv, wlx_ref[...]) + _dg(msg, wlm_ref[...]) + bl_ref[...]
        xr_ref[...] = _dg(xv, wrx_ref[...]) + _dg(msg, wrm_ref[...]) + br_ref[...]

    full = lambda shp: pl.BlockSpec(shp, lambda i: (0,) * len(shp))
    return pl.pallas_call(
        body,
        grid=(nb,),
        in_specs=[
            pl.BlockSpec((RB, f_in), lambda i: (i, 0)),
            pl.BlockSpec((NC, RB, acc0.shape[2]), lambda i: (0, i, 0)),
            full((1, hid)), full((1, hid)),
            full((d_out, f_in)), full((d_out, hid)), full((1, d_out)),
            full((d_out, f_in)), full((d_out, hid)), full((1, d_out)),
        ],
        out_specs=[
            pl.BlockSpec((RB, d_out), lambda i: (i, 0)),
            pl.BlockSpec((RB, d_out), lambda i: (i, 0)),
        ],
        out_shape=[
            jax.ShapeDtypeStruct((n, d_out), _f32),
            jax.ShapeDtypeStruct((n, d_out), _f32),
        ],
    )(x, acc0, we_row, be_row, wlx, wlm, bl, wrx, wrm, br)


def _tc_mid(acc1a, acc1b, xl1, xr1, att_row, bias_row, wl2, bl2, wr2, br2):
    n, d = xl1.shape
    h_ = 4
    c_ = d // h_
    wh = d // 2 + 2
    nb = n // RB
    d2 = wl2.shape[0]

    def body(a0_ref, a1_ref, xl_ref, xr_ref, att_ref, bias_ref, wl2_ref,
             bl2_ref, wr2_ref, br2_ref, xl2_ref, xr2_ref):
        dh = d // 2
        num = jnp.concatenate(
            [a0_ref[0, :, 0:dh] + a0_ref[1, :, 0:dh],
             a1_ref[0, :, 0:dh] + a1_ref[1, :, 0:dh]], axis=1)
        den = jnp.concatenate(
            [a0_ref[0, :, dh:dh + 2] + a0_ref[1, :, dh:dh + 2],
             a1_ref[0, :, dh:dh + 2] + a1_ref[1, :, dh:dh + 2]], axis=1)
        xlv = xl_ref[...]
        m = xlv + xr_ref[...]
        m = jnp.maximum(m, 0.2 * m)
        t = m * att_ref[...]
        ci = lax.broadcasted_iota(jnp.int32, (d, h_), 0) // c_
        hi = lax.broadcasted_iota(jnp.int32, (d, h_), 1)
        sel = (ci == hi).astype(_f32)
        als = lax.dot_general(t, sel, (((1,), (0,)), ((), ())),
                              preferred_element_type=_f32)
        exs = jnp.exp(als)
        den = den + exs
        ex128 = _dg(exs, sel)
        den128 = _dg(den, sel)
        out = (num + ex128 * xlv) / (den128 + 1e-16) + bias_ref[...]
        h1 = jnp.where(out > 0, out, jnp.exp(out) - 1.0)
        xl2_ref[...] = _dg(h1, wl2_ref[...]) + bl2_ref[...]
        xr2_ref[...] = _dg(h1, wr2_ref[...]) + br2_ref[...]

    full = lambda shp: pl.BlockSpec(shp, lambda i: (0,) * len(shp))
    return pl.pallas_call(
        body,
        grid=(nb,),
        in_specs=[
            pl.BlockSpec((NC, RB, wh), lambda i: (0, i, 0)),
            pl.BlockSpec((NC, RB, wh), lambda i: (0, i, 0)),
            pl.BlockSpec((RB, d), lambda i: (i, 0)),
            pl.BlockSpec((RB, d), lambda i: (i, 0)),
            full((1, d)), full((1, d)),
            full((d2, d)), full((1, d2)),
            full((d2, d)), full((1, d2)),
        ],
        out_specs=[
            pl.BlockSpec((RB, d2), lambda i: (i, 0)),
            pl.BlockSpec((RB, d2), lambda i: (i, 0)),
        ],
        out_shape=[
            jax.ShapeDtypeStruct((n, d2), _f32),
            jax.ShapeDtypeStruct((n, d2), _f32),
        ],
    )(acc1a, acc1b, xl1, xr1, att_row, bias_row, wl2, bl2, wr2, br2)


def _tc_post(acc2, xl2, xr2, att_row, bias_row, batchf, wf, bf, num_graphs):
    n, d = xl2.shape
    w = d + 1
    nb = n // RB
    ncls = wf.shape[0]

    def body(a_ref, xl_ref, xr_ref, att_ref, bias_ref, b_ref, wf_ref, bf_ref,
             out_ref, sums_ref, cnts_ref):
        i = pl.program_id(0)

        @pl.when(i == 0)
        def _():
            sums_ref[...] = jnp.zeros_like(sums_ref)
            cnts_ref[...] = jnp.zeros_like(cnts_ref)

        num = a_ref[0, :, 0:d] + a_ref[1, :, 0:d]
        den = a_ref[0, :, d:d + 1] + a_ref[1, :, d:d + 1]
        xlv = xl_ref[...]
        m = xlv + xr_ref[...]
        m = jnp.maximum(m, 0.2 * m)
        t = m * att_ref[...]
        al = jnp.sum(t, axis=1, keepdims=True)
        exs = jnp.exp(al)
        out = (num + exs * xlv) / (den + exs + 1e-16) + bias_ref[...]
        h2 = jnp.where(out > 0, out, jnp.exp(out) - 1.0)
        gi = lax.broadcasted_iota(jnp.int32, (RB, num_graphs), 1).astype(_f32)
        on = (b_ref[...] == gi).astype(_f32)
        sums_ref[...] += lax.dot_general(on, h2, (((0,), (0,)), ((), ())),
                                         preferred_element_type=_f32)
        cnts_ref[...] += lax.dot_general(on, jnp.ones_like(h2),
                                         (((0,), (0,)), ((), ())),
                                         preferred_element_type=_f32)

        @pl.when(i == nb - 1)
        def _():
            pooled = sums_ref[...] / jnp.maximum(cnts_ref[...], 1.0)
            logits = _dg(pooled, wf_ref[...]) + bf_ref[...]
            mx = jnp.max(logits, axis=1, keepdims=True)
            lse = mx + jnp.log(jnp.sum(jnp.exp(logits - mx), axis=1,
                                       keepdims=True))
            out_ref[...] = logits - lse

    full = lambda shp: pl.BlockSpec(shp, lambda i: (0,) * len(shp))
    return pl.pallas_call(
        body,
        grid=(nb,),
        in_specs=[
            pl.BlockSpec((NC, RB, w), lambda i: (0, i, 0)),
            pl.BlockSpec((RB, d), lambda i: (i, 0)),
            pl.BlockSpec((RB, d), lambda i: (i, 0)),
            full((1, d)), full((1, d)),
            pl.BlockSpec((RB, 1), lambda i: (i, 0)),
            full((ncls, d)), full((1, ncls)),
        ],
        out_specs=pl.BlockSpec((num_graphs, ncls), lambda i: (0, 0)),
        out_shape=jax.ShapeDtypeStruct((num_graphs, ncls), _f32),
        scratch_shapes=[
            pltpu.VMEM((num_graphs, d), _f32),
            pltpu.VMEM((num_graphs, d), _f32),
        ],
    )(acc2, xl2, xr2, att_row, bias_row, batchf, wf, bf)


def kernel(x, edge_index, edge_attr, batch, We, be, Wl1, bl1, Wr1, br1, att1,
           bias1, Wl2, bl2, Wr2, br2, att2, bias2, Wf, bf):
    n, f_in = x.shape
    src = edge_index[0]
    dst = edge_index[1]
    hid = We.shape[0]
    num_graphs = 64

    acc0 = _sc_scalar_scatter(dst, edge_attr, n)
    xl1, xr1 = _tc_pre(
        x, acc0,
        We[:, 0].reshape(1, hid), be.reshape(1, hid),
        Wl1[:, :f_in], Wl1[:, f_in:], bl1.reshape(1, -1),
        Wr1[:, :f_in], Wr1[:, f_in:], br1.reshape(1, -1),
    )
    src2d = src.reshape(-1, EB)
    dst2d = dst.reshape(-1, EB)
    att1f = att1.reshape(-1)
    dh = att1f.shape[0] // 2
    acc1a = _sc_edge_pass(xl1[:, :dh], xr1[:, :dh], src2d, dst2d,
                          att1f[:dh], heads=2)
    acc1b = _sc_edge_pass(xl1[:, dh:], xr1[:, dh:], src2d, dst2d,
                          att1f[dh:], heads=2)
    xl2, xr2 = _tc_mid(
        acc1a, acc1b, xl1, xr1, att1.reshape(1, -1), bias1.reshape(1, -1),
        Wl2, bl2.reshape(1, -1), Wr2, br2.reshape(1, -1),
    )
    acc2 = _sc_edge_pass(xl2, xr2, src2d, dst2d, att2.reshape(-1), heads=1)
    return _tc_post(
        acc2, xl2, xr2, att2.reshape(1, -1), bias2.reshape(1, -1),
        batch.astype(_f32).reshape(-1, 1), Wf, bf.reshape(1, -1), num_graphs,
    )


# speedup vs baseline: 5.8817x; 4.9580x over previous
import dataclasses
import functools

import jax
import jax.numpy as jnp
from jax import lax
from jax.experimental import pallas as pl
from jax.experimental.pallas import tpu as pltpu
from jax.experimental.pallas import tpu_sc as plsc

NC, NS, L = 2, 16, 16
NW = NC * NS
EB = 128
RB = 1000

_f32 = jnp.float32


def _vsc_mesh():
    return plsc.VectorSubcoreMesh(
        core_axis_name="c", subcore_axis_name="s", num_cores=NC, num_subcores=NS
    )


def _sc_params():
    cp = pltpu.CompilerParams()
    fields = pltpu.CompilerParams.__dataclass_fields__
    if "needs_layout_passes" in fields:
        cp = dataclasses.replace(cp, needs_layout_passes=False)
    if "use_tc_tiling_on_sc" in fields:
        cp = dataclasses.replace(cp, use_tc_tiling_on_sc=False)
    return cp


def _iota16():
    return lax.iota(jnp.int32, L)


def _zero_rows(ref, nrows, width):
    nfull = width // L
    tail = width - nfull * L

    @pl.loop(0, nrows)
    def _(r):
        for q in range(nfull):
            ref[r, pl.ds(q * L, L)] = jnp.zeros((L,), _f32)
        if tail:
            rid = jnp.full((L,), r, jnp.int32)
            colt = jnp.minimum(nfull * L + _iota16(), width - 1)
            plsc.store_scatter(ref, [rid, colt], jnp.zeros((L,), _f32),
                               mask=_iota16() < tail)


def _sc_scalar_scatter(dst, edge_attr, n_nodes):
    e_total = dst.shape[0]
    nblk = e_total // EB
    w0 = 8
    rows_per = (n_nodes // NS) & ~7
    rem = n_nodes - rows_per * NS
    zr = 48

    @functools.partial(
        pl.kernel,
        out_type=jax.ShapeDtypeStruct((NC, n_nodes, w0), _f32),
        mesh=_vsc_mesh(),
        scratch_types=[
            pltpu.VMEM((EB,), jnp.int32),
            pltpu.VMEM((EB,), _f32),
            pltpu.VMEM((EB, w0), _f32),
            pltpu.VMEM((zr, w0), _f32),
            pltpu.VMEM_SHARED((n_nodes, w0), _f32),
        ],
        compiler_params=_sc_params(),
    )
    def k(dst_hbm, ea_hbm, out_hbm, di, ea_v, rows_v, zb, acc):
        cid = lax.axis_index("c")
        sid = lax.axis_index("s")
        wid = sid * NC + cid

        _zero_rows(zb, zr, w0)
        _zero_rows(rows_v, EB, w0)

        @pl.loop(0, EB // L)
        def _(g):
            rid = g * L + _iota16()
            cid1 = jnp.full((L,), 1, jnp.int32)
            plsc.store_scatter(rows_v, [rid, cid1], jnp.ones((L,), _f32))

        @pl.loop(0, rows_per, step=zr)
        def _(j):
            pltpu.sync_copy(zb, acc.at[pl.ds(sid * rows_per + j, zr)])

        @pl.when(sid == NS - 1)
        def _():
            pltpu.sync_copy(zb.at[pl.ds(0, rem)],
                            acc.at[pl.ds(NS * rows_per, rem)])

        plsc.subcore_barrier()

        @pl.loop(wid, nblk, step=NW)
        def _(b):
            base = b * EB
            pltpu.sync_copy(dst_hbm.at[pl.ds(base, EB)], di)
            pltpu.sync_copy(ea_hbm.at[pl.ds(base, EB)], ea_v)

            @pl.loop(0, EB // L)
            def _(g):
                rid = g * L + _iota16()
                cid0 = jnp.zeros((L,), jnp.int32)
                plsc.store_scatter(rows_v, [rid, cid0], ea_v[pl.ds(g * L, L)])

            pltpu.sync_copy(rows_v, acc.at[di], add=True)

        plsc.subcore_barrier()
        pltpu.sync_copy(
            acc.at[pl.ds(sid * rows_per, rows_per)],
            out_hbm.at[cid, pl.ds(sid * rows_per, rows_per)],
        )

        @pl.when(sid == NS - 1)
        def _():
            pltpu.sync_copy(
                acc.at[pl.ds(NS * rows_per, rem)],
                out_hbm.at[cid, pl.ds(NS * rows_per, rem)],
            )

    return k(dst, edge_attr)


def _sc_edge_pass(xl, xr, src, dst, attf, heads):
    n_nodes, d = xl.shape
    h_ = heads
    c_ = d // h_
    w = d + h_
    nblk = src.shape[0]
    nb_main = nblk // NW
    tail = nblk - nb_main * NW
    rows_per = (n_nodes // NS) & ~7
    rem = n_nodes - rows_per * NS
    zr = 48

    @functools.partial(
        pl.kernel,
        out_type=jax.ShapeDtypeStruct((NC, n_nodes, w), _f32),
        mesh=_vsc_mesh(),
        scratch_types=[
            pltpu.VMEM((nb_main + 1, EB), jnp.int32),
            pltpu.VMEM((nb_main + 1, EB), jnp.int32),
            pltpu.VMEM((EB, d), _f32),
            pltpu.VMEM((EB, d), _f32),
            pltpu.VMEM((EB, d), _f32),
            pltpu.VMEM((EB, d), _f32),
            pltpu.VMEM((EB, w), _f32),
            pltpu.VMEM((EB, w), _f32),
            pltpu.VMEM((d,), _f32),
            pltpu.VMEM((zr, w), _f32),
            pltpu.VMEM_SHARED((n_nodes, w), _f32),
            pltpu.SemaphoreType.DMA,
            pltpu.SemaphoreType.DMA,
            pltpu.SemaphoreType.DMA,
            pltpu.SemaphoreType.DMA,
        ],
        compiler_params=_sc_params(),
    )
    def k(xl_hbm, xr_hbm, src_hbm, dst_hbm, att_hbm, out_hbm,
          silo, dilo, xlb0, xlb1, xrb0, xrb1, ob0, ob1, att_v, zb, acc,
          gsem0, gsem1, ssem0, ssem1):
        cid = lax.axis_index("c")
        sid = lax.axis_index("s")
        wid = sid * NC + cid
        bufs = ((xlb0, xrb0, ob0, gsem0, ssem0),
                (xlb1, xrb1, ob1, gsem1, ssem1))

        pltpu.sync_copy(att_hbm, att_v)
        _zero_rows(zb, zr, w)

        pltpu.sync_copy(src_hbm.at[pl.ds(wid * nb_main, nb_main)],
                        silo.at[pl.ds(0, nb_main)])
        pltpu.sync_copy(dst_hbm.at[pl.ds(wid * nb_main, nb_main)],
                        dilo.at[pl.ds(0, nb_main)])
        if tail:
            @pl.when(wid < tail)
            def _():
                pltpu.sync_copy(src_hbm.at[NW * nb_main + wid],
                                silo.at[nb_main])
                pltpu.sync_copy(dst_hbm.at[NW * nb_main + wid],
                                dilo.at[nb_main])

        @pl.loop(0, rows_per, step=zr)
        def _(j):
            pltpu.sync_copy(zb, acc.at[pl.ds(sid * rows_per + j, zr)])

        @pl.when(sid == NS - 1)
        def _():
            pltpu.sync_copy(zb.at[pl.ds(0, rem)],
                            acc.at[pl.ds(NS * rows_per, rem)])

        plsc.subcore_barrier()

        def issue_gather(kb, p):
            xlb_, xrb_, _, gsem_, _ = bufs[p]
            pltpu.async_copy(xl_hbm.at[silo.at[kb]], xlb_, gsem_)
            pltpu.async_copy(xr_hbm.at[dilo.at[kb]], xrb_, gsem_)

        def wait_gather(kb, p):
            xlb_, xrb_, _, gsem_, _ = bufs[p]
            pltpu.make_async_copy(xl_hbm.at[silo.at[kb]], xlb_, gsem_).wait()
            pltpu.make_async_copy(xr_hbm.at[dilo.at[kb]], xrb_, gsem_).wait()

        def issue_scatter(kb, p):
            _, _, ob_, _, ssem_ = bufs[p]
            pltpu.async_copy(ob_, acc.at[dilo.at[kb]], ssem_, add=True)

        def wait_scatter(kb, p):
            _, _, ob_, _, ssem_ = bufs[p]
            pltpu.make_async_copy(ob_, acc.at[dilo.at[kb]], ssem_).wait()

        def compute(p):
            xlb_, xrb_, ob_, _, _ = bufs[p]
            attv = [att_v[pl.ds(j * L, L)] for j in range(d // L)]
            lane = _iota16()
            mask0 = lane == 0
            nv = c_ // L

            @plsc.parallel_loop(0, EB, unroll=2)
            def _(e):
                for h in range(h_):
                    t = None
                    for u in range(nv):
                        j = h * nv + u
                        ds_ = pl.ds(j * L, L)
                        m = xlb_[e, ds_] + xrb_[e, ds_]
                        m = jnp.maximum(m, 0.2 * m)
                        tu = m * attv[j]
                        t = tu if t is None else t + tu
                    for s in (1, 2, 4, 8):
                        t = t + t.at[lane ^ s].get(mode="promise_in_bounds")
                    ex = jnp.exp(t)
                    plsc.store_scatter(
                        ob_, [jnp.full((L,), e, jnp.int32),
                              jnp.full((L,), d + h, jnp.int32)],
                        ex, mask=mask0)
                    for u in range(nv):
                        j = h * nv + u
                        ds_ = pl.ds(j * L, L)
                        ob_[e, ds_] = xlb_[e, ds_] * ex

        issue_gather(0, 0)

        @pl.loop(0, nb_main // 2)
        def _(j):
            for p in (0, 1):
                kb = 2 * j + p
                if p == 0:
                    issue_gather(kb + 1, 1)
                else:
                    @pl.when(kb + 1 < nb_main)
                    def _():
                        issue_gather(kb + 1, 0)
                wait_gather(kb, p)

                @pl.when(kb >= 2)
                def _():
                    wait_scatter(kb - 2, p)

                compute(p)
                issue_scatter(kb, p)

        wait_scatter(nb_main - 2, 0)
        wait_scatter(nb_main - 1, 1)

        if tail:
            @pl.when(wid < tail)
            def _():
                issue_gather(nb_main, 0)
                wait_gather(nb_main, 0)
                compute(0)
                pltpu.sync_copy(ob0, acc.at[dilo.at[nb_main]], add=True)

        plsc.subcore_barrier()
        pltpu.sync_copy(
            acc.at[pl.ds(sid * rows_per, rows_per)],
            out_hbm.at[cid, pl.ds(sid * rows_per, rows_per)],
        )

        @pl.when(sid == NS - 1)
        def _():
            pltpu.sync_copy(
                acc.at[pl.ds(NS * rows_per, rem)],
                out_hbm.at[cid, pl.ds(NS * rows_per, rem)],
            )

    return k(xl, xr, src, dst, attf)


def _dg(a, b):
    return lax.dot_general(a, b, (((1,), (1,)), ((), ())),
                           preferred_element_type=_f32)


def _tc_pre(x, acc0, we_row, be_row, wlx, wlm, bl, wrx, wrm, br):
    n, f_in = x.shape
    nb = n // RB
    hid = we_row.shape[1]
    d_out = bl.shape[1]

    def body(x_ref, a_ref, we_ref, be_ref, wlx_ref, wlm_ref, bl_ref,
             wrx_ref, wrm_ref, br_ref, xl_ref, xr_ref):
        s = a_ref[0, :, 0:1] + a_ref[1, :, 0:1]
        c = a_ref[0, :, 1:2] + a_ref[1, :, 1:2]
        msg = s * we_ref[...] + c * be_ref[...]
        xv = x_ref[...]
        xl_ref[...] = _dg(xv, wlx_ref[...]) + _dg(msg, wlm_ref[...]) + bl_ref[...]
        xr_ref[...] = _dg(xv, wrx_ref[...]) + _dg(msg, wrm_ref[...]) + br_ref[...]

    full = lambda shp: pl.BlockSpec(shp, lambda i: (0,) * len(shp))
    return pl.pallas_call(
        body,
        grid=(nb,),
        in_specs=[
            pl.BlockSpec((RB, f_in), lambda i: (i, 0)),
            pl.BlockSpec((NC, RB, acc0.shape[2]), lambda i: (0, i, 0)),
            full((1, hid)), full((1, hid)),
            full((d_out, f_in)), full((d_out, hid)), full((1, d_out)),
            full((d_out, f_in)), full((d_out, hid)), full((1, d_out)),
        ],
        out_specs=[
            pl.BlockSpec((RB, d_out), lambda i: (i, 0)),
            pl.BlockSpec((RB, d_out), lambda i: (i, 0)),
        ],
        out_shape=[
            jax.ShapeDtypeStruct((n, d_out), _f32),
            jax.ShapeDtypeStruct((n, d_out), _f32),
        ],
    )(x, acc0, we_row, be_row, wlx, wlm, bl, wrx, wrm, br)


def _tc_mid(acc1a, acc1b, xl1, xr1, att_row, bias_row, wl2, bl2, wr2, br2):
    n, d = xl1.shape
    h_ = 4
    c_ = d // h_
    wh = d // 2 + 2
    nb = n // RB
    d2 = wl2.shape[0]

    def body(a0_ref, a1_ref, xl_ref, xr_ref, att_ref, bias_ref, wl2_ref,
             bl2_ref, wr2_ref, br2_ref, xl2_ref, xr2_ref):
        dh = d // 2
        num = jnp.concatenate(
            [a0_ref[0, :, 0:dh] + a0_ref[1, :, 0:dh],
             a1_ref[0, :, 0:dh] + a1_ref[1, :, 0:dh]], axis=1)
        den = jnp.concatenate(
            [a0_ref[0, :, dh:dh + 2] + a0_ref[1, :, dh:dh + 2],
             a1_ref[0, :, dh:dh + 2] + a1_ref[1, :, dh:dh + 2]], axis=1)
        xlv = xl_ref[...]
        m = xlv + xr_ref[...]
        m = jnp.maximum(m, 0.2 * m)
        t = m * att_ref[...]
        ci = lax.broadcasted_iota(jnp.int32, (d, h_), 0) // c_
        hi = lax.broadcasted_iota(jnp.int32, (d, h_), 1)
        sel = (ci == hi).astype(_f32)
        als = lax.dot_general(t, sel, (((1,), (0,)), ((), ())),
                              preferred_element_type=_f32)
        exs = jnp.exp(als)
        den = den + exs
        ex128 = _dg(exs, sel)
        den128 = _dg(den, sel)
        out = (num + ex128 * xlv) / (den128 + 1e-16) + bias_ref[...]
        h1 = jnp.where(out > 0, out, jnp.exp(out) - 1.0)
        xl2_ref[...] = _dg(h1, wl2_ref[...]) + bl2_ref[...]
        xr2_ref[...] = _dg(h1, wr2_ref[...]) + br2_ref[...]

    full = lambda shp: pl.BlockSpec(shp, lambda i: (0,) * len(shp))
    return pl.pallas_call(
        body,
        grid=(nb,),
        in_specs=[
            pl.BlockSpec((NC, RB, wh), lambda i: (0, i, 0)),
            pl.BlockSpec((NC, RB, wh), lambda i: (0, i, 0)),
            pl.BlockSpec((RB, d), lambda i: (i, 0)),
            pl.BlockSpec((RB, d), lambda i: (i, 0)),
            full((1, d)), full((1, d)),
            full((d2, d)), full((1, d2)),
            full((d2, d)), full((1, d2)),
        ],
        out_specs=[
            pl.BlockSpec((RB, d2), lambda i: (i, 0)),
            pl.BlockSpec((RB, d2), lambda i: (i, 0)),
        ],
        out_shape=[
            jax.ShapeDtypeStruct((n, d2), _f32),
            jax.ShapeDtypeStruct((n, d2), _f32),
        ],
    )(acc1a, acc1b, xl1, xr1, att_row, bias_row, wl2, bl2, wr2, br2)


def _tc_post(acc2, xl2, xr2, att_row, bias_row, batchf, wf, bf, num_graphs):
    n, d = xl2.shape
    w = d + 1
    nb = n // RB
    ncls = wf.shape[0]

    def body(a_ref, xl_ref, xr_ref, att_ref, bias_ref, b_ref, wf_ref, bf_ref,
             out_ref, sums_ref, cnts_ref):
        i = pl.program_id(0)

        @pl.when(i == 0)
        def _():
            sums_ref[...] = jnp.zeros_like(sums_ref)
            cnts_ref[...] = jnp.zeros_like(cnts_ref)

        num = a_ref[0, :, 0:d] + a_ref[1, :, 0:d]
        den = a_ref[0, :, d:d + 1] + a_ref[1, :, d:d + 1]
        xlv = xl_ref[...]
        m = xlv + xr_ref[...]
        m = jnp.maximum(m, 0.2 * m)
        t = m * att_ref[...]
        al = jnp.sum(t, axis=1, keepdims=True)
        exs = jnp.exp(al)
        out = (num + exs * xlv) / (den + exs + 1e-16) + bias_ref[...]
        h2 = jnp.where(out > 0, out, jnp.exp(out) - 1.0)
        gi = lax.broadcasted_iota(jnp.int32, (RB, num_graphs), 1).astype(_f32)
        on = (b_ref[...] == gi).astype(_f32)
        sums_ref[...] += lax.dot_general(on, h2, (((0,), (0,)), ((), ())),
                                         preferred_element_type=_f32)
        cnts_ref[...] += lax.dot_general(on, jnp.ones_like(h2),
                                         (((0,), (0,)), ((), ())),
                                         preferred_element_type=_f32)

        @pl.when(i == nb - 1)
        def _():
            pooled = sums_ref[...] / jnp.maximum(cnts_ref[...], 1.0)
            logits = _dg(pooled, wf_ref[...]) + bf_ref[...]
            mx = jnp.max(logits, axis=1, keepdims=True)
            lse = mx + jnp.log(jnp.sum(jnp.exp(logits - mx), axis=1,
                                       keepdims=True))
            out_ref[...] = logits - lse

    full = lambda shp: pl.BlockSpec(shp, lambda i: (0,) * len(shp))
    return pl.pallas_call(
        body,
        grid=(nb,),
        in_specs=[
            pl.BlockSpec((NC, RB, w), lambda i: (0, i, 0)),
            pl.BlockSpec((RB, d), lambda i: (i, 0)),
            pl.BlockSpec((RB, d), lambda i: (i, 0)),
            full((1, d)), full((1, d)),
            pl.BlockSpec((RB, 1), lambda i: (i, 0)),
            full((ncls, d)), full((1, ncls)),
        ],
        out_specs=pl.BlockSpec((num_graphs, ncls), lambda i: (0, 0)),
        out_shape=jax.ShapeDtypeStruct((num_graphs, ncls), _f32),
        scratch_shapes=[
            pltpu.VMEM((num_graphs, d), _f32),
            pltpu.VMEM((num_graphs, d), _f32),
        ],
    )(acc2, xl2, xr2, att_row, bias_row, batchf, wf, bf)


def kernel(x, edge_index, edge_attr, batch, We, be, Wl1, bl1, Wr1, br1, att1,
           bias1, Wl2, bl2, Wr2, br2, att2, bias2, Wf, bf):
    n, f_in = x.shape
    src = edge_index[0]
    dst = edge_index[1]
    hid = We.shape[0]
    num_graphs = 64

    acc0 = _sc_scalar_scatter(dst, edge_attr, n)
    xl1, xr1 = _tc_pre(
        x, acc0,
        We[:, 0].reshape(1, hid), be.reshape(1, hid),
        Wl1[:, :f_in], Wl1[:, f_in:], bl1.reshape(1, -1),
        Wr1[:, :f_in], Wr1[:, f_in:], br1.reshape(1, -1),
    )
    src2d = src.reshape(-1, EB)
    dst2d = dst.reshape(-1, EB)
    att1f = att1.reshape(-1)
    dh = att1f.shape[0] // 2
    acc1a = _sc_edge_pass(xl1[:, :dh], xr1[:, :dh], src2d, dst2d,
                          att1f[:dh], heads=2)
    acc1b = _sc_edge_pass(xl1[:, dh:], xr1[:, dh:], src2d, dst2d,
                          att1f[dh:], heads=2)
    xl2, xr2 = _tc_mid(
        acc1a, acc1b, xl1, xr1, att1.reshape(1, -1), bias1.reshape(1, -1),
        Wl2, bl2.reshape(1, -1), Wr2, br2.reshape(1, -1),
    )
    acc2 = _sc_edge_pass(xl2, xr2, src2d, dst2d, att2.reshape(-1), heads=1)
    return _tc_post(
        acc2, xl2, xr2, att2.reshape(1, -1), bias2.reshape(1, -1),
        batch.astype(_f32).reshape(-1, 1), Wf, bf.reshape(1, -1), num_graphs,
    )


# speedup vs baseline: 6.7943x; 1.1551x over previous
import dataclasses
import functools

import jax
import jax.numpy as jnp
from jax import lax
from jax.experimental import pallas as pl
from jax.experimental.pallas import tpu as pltpu
from jax.experimental.pallas import tpu_sc as plsc

NC, NS, L = 2, 16, 16
NW = NC * NS
EB = 128
RB = 1000

_f32 = jnp.float32


def _vsc_mesh():
    return plsc.VectorSubcoreMesh(
        core_axis_name="c", subcore_axis_name="s", num_cores=NC, num_subcores=NS
    )


def _sc_params():
    cp = pltpu.CompilerParams()
    fields = pltpu.CompilerParams.__dataclass_fields__
    if "needs_layout_passes" in fields:
        cp = dataclasses.replace(cp, needs_layout_passes=False)
    if "use_tc_tiling_on_sc" in fields:
        cp = dataclasses.replace(cp, use_tc_tiling_on_sc=False)
    return cp


def _iota16():
    return lax.iota(jnp.int32, L)


def _zero_rows(ref, nrows, width):
    nfull = width // L
    tail = width - nfull * L

    @pl.loop(0, nrows)
    def _(r):
        for q in range(nfull):
            ref[r, pl.ds(q * L, L)] = jnp.zeros((L,), _f32)
        if tail:
            rid = jnp.full((L,), r, jnp.int32)
            colt = jnp.minimum(nfull * L + _iota16(), width - 1)
            plsc.store_scatter(ref, [rid, colt], jnp.zeros((L,), _f32),
                               mask=_iota16() < tail)


def _sc_scalar_scatter(dst, edge_attr, n_nodes):
    nblk = dst.shape[0]
    w0 = 8
    nb_main = nblk // NW
    tail = nblk - nb_main * NW
    rows_per = (n_nodes // NS) & ~7
    rem = n_nodes - rows_per * NS
    zr = 48

    @functools.partial(
        pl.kernel,
        out_type=jax.ShapeDtypeStruct((NC, n_nodes, w0), _f32),
        mesh=_vsc_mesh(),
        scratch_types=[
            pltpu.VMEM((nb_main + 1, EB), jnp.int32),
            pltpu.VMEM((nb_main + 1, EB), _f32),
            pltpu.VMEM((EB, w0), _f32),
            pltpu.VMEM((EB, w0), _f32),
            pltpu.VMEM((zr, w0), _f32),
            pltpu.VMEM_SHARED((n_nodes, w0), _f32),
            pltpu.SemaphoreType.DMA,
            pltpu.SemaphoreType.DMA,
        ],
        compiler_params=_sc_params(),
    )
    def k(dst_hbm, ea_hbm, out_hbm, dilo, ealo, rv0, rv1, zb, acc,
          ssem0, ssem1):
        cid = lax.axis_index("c")
        sid = lax.axis_index("s")
        wid = sid * NC + cid
        rvs = ((rv0, ssem0), (rv1, ssem1))

        _zero_rows(zb, zr, w0)
        _zero_rows(rv0, EB, w0)
        _zero_rows(rv1, EB, w0)

        for rv in (rv0, rv1):
            @pl.loop(0, EB // L)
            def _(g, rv=rv):
                rid = g * L + _iota16()
                cid1 = jnp.full((L,), 1, jnp.int32)
                plsc.store_scatter(rv, [rid, cid1], jnp.ones((L,), _f32))

        pltpu.sync_copy(dst_hbm.at[pl.ds(wid * nb_main, nb_main)],
                        dilo.at[pl.ds(0, nb_main)])
        pltpu.sync_copy(ea_hbm.at[pl.ds(wid * nb_main, nb_main)],
                        ealo.at[pl.ds(0, nb_main)])
        if tail:
            @pl.when(wid < tail)
            def _():
                pltpu.sync_copy(dst_hbm.at[NW * nb_main + wid],
                                dilo.at[nb_main])
                pltpu.sync_copy(ea_hbm.at[NW * nb_main + wid],
                                ealo.at[nb_main])

        @pl.loop(0, rows_per, step=zr)
        def _(j):
            pltpu.sync_copy(zb, acc.at[pl.ds(sid * rows_per + j, zr)])

        @pl.when(sid == NS - 1)
        def _():
            pltpu.sync_copy(zb.at[pl.ds(0, rem)],
                            acc.at[pl.ds(NS * rows_per, rem)])

        plsc.subcore_barrier()

        def fill(kb, p):
            rv, _ = rvs[p]

            @pl.loop(0, EB // L)
            def _(g):
                rid = g * L + _iota16()
                cid0 = jnp.zeros((L,), jnp.int32)
                plsc.store_scatter(rv, [rid, cid0],
                                   ealo[kb, pl.ds(g * L, L)])

        def issue_scatter(kb, p):
            rv, ssem_ = rvs[p]
            pltpu.async_copy(rv, acc.at[dilo.at[kb]], ssem_, add=True)

        def wait_scatter(kb, p):
            rv, ssem_ = rvs[p]
            pltpu.make_async_copy(rv, acc.at[dilo.at[kb]], ssem_).wait()

        @pl.loop(0, nb_main // 2)
        def _(j):
            for p in (0, 1):
                kb = 2 * j + p

                @pl.when(kb >= 2)
                def _():
                    wait_scatter(kb - 2, p)

                fill(kb, p)
                issue_scatter(kb, p)

        wait_scatter(nb_main - 2, 0)
        wait_scatter(nb_main - 1, 1)

        if tail:
            @pl.when(wid < tail)
            def _():
                fill(nb_main, 0)
                pltpu.sync_copy(rv0, acc.at[dilo.at[nb_main]], add=True)

        plsc.subcore_barrier()
        pltpu.sync_copy(
            acc.at[pl.ds(sid * rows_per, rows_per)],
            out_hbm.at[cid, pl.ds(sid * rows_per, rows_per)],
        )

        @pl.when(sid == NS - 1)
        def _():
            pltpu.sync_copy(
                acc.at[pl.ds(NS * rows_per, rem)],
                out_hbm.at[cid, pl.ds(NS * rows_per, rem)],
            )

    return k(dst, edge_attr)


def _sc_edge_pass(xl, xr, src, dst, attf, heads):
    n_nodes, d = xl.shape
    h_ = heads
    c_ = d // h_
    w = d + h_
    nblk = src.shape[0]
    nb_main = nblk // NW
    tail = nblk - nb_main * NW
    rows_per = (n_nodes // NS) & ~7
    rem = n_nodes - rows_per * NS
    zr = 48

    @functools.partial(
        pl.kernel,
        out_type=jax.ShapeDtypeStruct((NC, n_nodes, w), _f32),
        mesh=_vsc_mesh(),
        scratch_types=[
            pltpu.VMEM((nb_main + 1, EB), jnp.int32),
            pltpu.VMEM((nb_main + 1, EB), jnp.int32),
            pltpu.VMEM((EB, d), _f32),
            pltpu.VMEM((EB, d), _f32),
            pltpu.VMEM((EB, d), _f32),
            pltpu.VMEM((EB, d), _f32),
            pltpu.VMEM((EB, w), _f32),
            pltpu.VMEM((EB, w), _f32),
            pltpu.VMEM((d,), _f32),
            pltpu.VMEM((zr, w), _f32),
            pltpu.VMEM_SHARED((n_nodes, w), _f32),
            pltpu.SemaphoreType.DMA,
            pltpu.SemaphoreType.DMA,
            pltpu.SemaphoreType.DMA,
            pltpu.SemaphoreType.DMA,
        ],
        compiler_params=_sc_params(),
    )
    def k(xl_hbm, xr_hbm, src_hbm, dst_hbm, att_hbm, out_hbm,
          silo, dilo, xlb0, xlb1, xrb0, xrb1, ob0, ob1, att_v, zb, acc,
          gsem0, gsem1, ssem0, ssem1):
        cid = lax.axis_index("c")
        sid = lax.axis_index("s")
        wid = sid * NC + cid
        bufs = ((xlb0, xrb0, ob0, gsem0, ssem0),
                (xlb1, xrb1, ob1, gsem1, ssem1))

        pltpu.sync_copy(att_hbm, att_v)
        _zero_rows(zb, zr, w)

        pltpu.sync_copy(src_hbm.at[pl.ds(wid * nb_main, nb_main)],
                        silo.at[pl.ds(0, nb_main)])
        pltpu.sync_copy(dst_hbm.at[pl.ds(wid * nb_main, nb_main)],
                        dilo.at[pl.ds(0, nb_main)])
        if tail:
            @pl.when(wid < tail)
            def _():
                pltpu.sync_copy(src_hbm.at[NW * nb_main + wid],
                                silo.at[nb_main])
                pltpu.sync_copy(dst_hbm.at[NW * nb_main + wid],
                                dilo.at[nb_main])

        @pl.loop(0, rows_per, step=zr)
        def _(j):
            pltpu.sync_copy(zb, acc.at[pl.ds(sid * rows_per + j, zr)])

        @pl.when(sid == NS - 1)
        def _():
            pltpu.sync_copy(zb.at[pl.ds(0, rem)],
                            acc.at[pl.ds(NS * rows_per, rem)])

        plsc.subcore_barrier()

        def issue_gather(kb, p):
            xlb_, xrb_, _, gsem_, _ = bufs[p]
            pltpu.async_copy(xl_hbm.at[silo.at[kb]], xlb_, gsem_)
            pltpu.async_copy(xr_hbm.at[dilo.at[kb]], xrb_, gsem_)

        def wait_gather(kb, p):
            xlb_, xrb_, _, gsem_, _ = bufs[p]
            pltpu.make_async_copy(xl_hbm.at[silo.at[kb]], xlb_, gsem_).wait()
            pltpu.make_async_copy(xr_hbm.at[dilo.at[kb]], xrb_, gsem_).wait()

        def issue_scatter(kb, p):
            _, _, ob_, _, ssem_ = bufs[p]
            pltpu.async_copy(ob_, acc.at[dilo.at[kb]], ssem_, add=True)

        def wait_scatter(kb, p):
            _, _, ob_, _, ssem_ = bufs[p]
            pltpu.make_async_copy(ob_, acc.at[dilo.at[kb]], ssem_).wait()

        def compute(p):
            xlb_, xrb_, ob_, _, _ = bufs[p]
            attv = [att_v[pl.ds(j * L, L)] for j in range(d // L)]
            lane = _iota16()
            mask0 = lane == 0
            nv = c_ // L

            @plsc.parallel_loop(0, EB, unroll=2)
            def _(e):
                for h in range(h_):
                    t = None
                    for u in range(nv):
                        j = h * nv + u
                        ds_ = pl.ds(j * L, L)
                        m = xlb_[e, ds_] + xrb_[e, ds_]
                        m = jnp.maximum(m, 0.2 * m)
                        tu = m * attv[j]
                        t = tu if t is None else t + tu
                    for s in (1, 2, 4, 8):
                        t = t + t.at[lane ^ s].get(mode="promise_in_bounds")
                    ex = jnp.exp(t)
                    plsc.store_scatter(
                        ob_, [jnp.full((L,), e, jnp.int32),
                              jnp.full((L,), d + h, jnp.int32)],
                        ex, mask=mask0)
                    for u in range(nv):
                        j = h * nv + u
                        ds_ = pl.ds(j * L, L)
                        ob_[e, ds_] = xlb_[e, ds_] * ex

        issue_gather(0, 0)

        @pl.loop(0, nb_main // 2)
        def _(j):
            for p in (0, 1):
                kb = 2 * j + p
                if p == 0:
                    issue_gather(kb + 1, 1)
                else:
                    @pl.when(kb + 1 < nb_main)
                    def _():
                        issue_gather(kb + 1, 0)
                wait_gather(kb, p)

                @pl.when(kb >= 2)
                def _():
                    wait_scatter(kb - 2, p)

                compute(p)
                issue_scatter(kb, p)

        wait_scatter(nb_main - 2, 0)
        wait_scatter(nb_main - 1, 1)

        if tail:
            @pl.when(wid < tail)
            def _():
                issue_gather(nb_main, 0)
                wait_gather(nb_main, 0)
                compute(0)
                pltpu.sync_copy(ob0, acc.at[dilo.at[nb_main]], add=True)

        plsc.subcore_barrier()
        pltpu.sync_copy(
            acc.at[pl.ds(sid * rows_per, rows_per)],
            out_hbm.at[cid, pl.ds(sid * rows_per, rows_per)],
        )

        @pl.when(sid == NS - 1)
        def _():
            pltpu.sync_copy(
                acc.at[pl.ds(NS * rows_per, rem)],
                out_hbm.at[cid, pl.ds(NS * rows_per, rem)],
            )

    return k(xl, xr, src, dst, attf)


def _dg(a, b):
    return lax.dot_general(a, b, (((1,), (1,)), ((), ())),
                           preferred_element_type=_f32)


def _tc_pre(x, acc0, we_row, be_row, wlx, wlm, bl, wrx, wrm, br):
    n, f_in = x.shape
    nb = n // RB
    hid = we_row.shape[1]
    d_out = bl.shape[1]

    def body(x_ref, a_ref, we_ref, be_ref, wlx_ref, wlm_ref, bl_ref,
             wrx_ref, wrm_ref, br_ref, xl_ref, xr_ref):
        s = a_ref[0, :, 0:1] + a_ref[1, :, 0:1]
        c = a_ref[0, :, 1:2] + a_ref[1, :, 1:2]
        msg = s * we_ref[...] + c * be_ref[...]
        xv = x_ref[...]
        xl_ref[...] = _dg(xv, wlx_ref[...]) + _dg(msg, wlm_ref[...]) + bl_ref[...]
        xr_ref[...] = _dg(xv, wrx_ref[...]) + _dg(msg, wrm_ref[...]) + br_ref[...]

    full = lambda shp: pl.BlockSpec(shp, lambda i: (0,) * len(shp))
    return pl.pallas_call(
        body,
        grid=(nb,),
        in_specs=[
            pl.BlockSpec((RB, f_in), lambda i: (i, 0)),
            pl.BlockSpec((NC, RB, acc0.shape[2]), lambda i: (0, i, 0)),
            full((1, hid)), full((1, hid)),
            full((d_out, f_in)), full((d_out, hid)), full((1, d_out)),
            full((d_out, f_in)), full((d_out, hid)), full((1, d_out)),
        ],
        out_specs=[
            pl.BlockSpec((RB, d_out), lambda i: (i, 0)),
            pl.BlockSpec((RB, d_out), lambda i: (i, 0)),
        ],
        out_shape=[
            jax.ShapeDtypeStruct((n, d_out), _f32),
            jax.ShapeDtypeStruct((n, d_out), _f32),
        ],
    )(x, acc0, we_row, be_row, wlx, wlm, bl, wrx, wrm, br)


def _tc_mid(acc1a, acc1b, xl1, xr1, att_row, bias_row, wl2, bl2, wr2, br2):
    n, d = xl1.shape
    h_ = 4
    c_ = d // h_
    wh = d // 2 + 2
    nb = n // RB
    d2 = wl2.shape[0]

    def body(a0_ref, a1_ref, xl_ref, xr_ref, att_ref, bias_ref, wl2_ref,
             bl2_ref, wr2_ref, br2_ref, xl2_ref, xr2_ref):
        dh = d // 2
        num = jnp.concatenate(
            [a0_ref[0, :, 0:dh] + a0_ref[1, :, 0:dh],
             a1_ref[0, :, 0:dh] + a1_ref[1, :, 0:dh]], axis=1)
        den = jnp.concatenate(
            [a0_ref[0, :, dh:dh + 2] + a0_ref[1, :, dh:dh + 2],
             a1_ref[0, :, dh:dh + 2] + a1_ref[1, :, dh:dh + 2]], axis=1)
        xlv = xl_ref[...]
        m = xlv + xr_ref[...]
        m = jnp.maximum(m, 0.2 * m)
        t = m * att_ref[...]
        ci = lax.broadcasted_iota(jnp.int32, (d, h_), 0) // c_
        hi = lax.broadcasted_iota(jnp.int32, (d, h_), 1)
        sel = (ci == hi).astype(_f32)
        als = lax.dot_general(t, sel, (((1,), (0,)), ((), ())),
                              preferred_element_type=_f32)
        exs = jnp.exp(als)
        den = den + exs
        ex128 = _dg(exs, sel)
        den128 = _dg(den, sel)
        out = (num + ex128 * xlv) / (den128 + 1e-16) + bias_ref[...]
        h1 = jnp.where(out > 0, out, jnp.exp(out) - 1.0)
        xl2_ref[...] = _dg(h1, wl2_ref[...]) + bl2_ref[...]
        xr2_ref[...] = _dg(h1, wr2_ref[...]) + br2_ref[...]

    full = lambda shp: pl.BlockSpec(shp, lambda i: (0,) * len(shp))
    return pl.pallas_call(
        body,
        grid=(nb,),
        in_specs=[
            pl.BlockSpec((NC, RB, wh), lambda i: (0, i, 0)),
            pl.BlockSpec((NC, RB, wh), lambda i: (0, i, 0)),
            pl.BlockSpec((RB, d), lambda i: (i, 0)),
            pl.BlockSpec((RB, d), lambda i: (i, 0)),
            full((1, d)), full((1, d)),
            full((d2, d)), full((1, d2)),
            full((d2, d)), full((1, d2)),
        ],
        out_specs=[
            pl.BlockSpec((RB, d2), lambda i: (i, 0)),
            pl.BlockSpec((RB, d2), lambda i: (i, 0)),
        ],
        out_shape=[
            jax.ShapeDtypeStruct((n, d2), _f32),
            jax.ShapeDtypeStruct((n, d2), _f32),
        ],
    )(acc1a, acc1b, xl1, xr1, att_row, bias_row, wl2, bl2, wr2, br2)


def _tc_post(acc2, xl2, xr2, att_row, bias_row, batchf, wf, bf, num_graphs):
    n, d = xl2.shape
    w = d + 1
    nb = n // RB
    ncls = wf.shape[0]

    def body(a_ref, xl_ref, xr_ref, att_ref, bias_ref, b_ref, wf_ref, bf_ref,
             out_ref, sums_ref, cnts_ref):
        i = pl.program_id(0)

        @pl.when(i == 0)
        def _():
            sums_ref[...] = jnp.zeros_like(sums_ref)
            cnts_ref[...] = jnp.zeros_like(cnts_ref)

        num = a_ref[0, :, 0:d] + a_ref[1, :, 0:d]
        den = a_ref[0, :, d:d + 1] + a_ref[1, :, d:d + 1]
        xlv = xl_ref[...]
        m = xlv + xr_ref[...]
        m = jnp.maximum(m, 0.2 * m)
        t = m * att_ref[...]
        al = jnp.sum(t, axis=1, keepdims=True)
        exs = jnp.exp(al)
        out = (num + exs * xlv) / (den + exs + 1e-16) + bias_ref[...]
        h2 = jnp.where(out > 0, out, jnp.exp(out) - 1.0)
        gi = lax.broadcasted_iota(jnp.int32, (RB, num_graphs), 1).astype(_f32)
        on = (b_ref[...] == gi).astype(_f32)
        sums_ref[...] += lax.dot_general(on, h2, (((0,), (0,)), ((), ())),
                                         preferred_element_type=_f32)
        cnts_ref[...] += lax.dot_general(on, jnp.ones_like(h2),
                                         (((0,), (0,)), ((), ())),
                                         preferred_element_type=_f32)

        @pl.when(i == nb - 1)
        def _():
            pooled = sums_ref[...] / jnp.maximum(cnts_ref[...], 1.0)
            logits = _dg(pooled, wf_ref[...]) + bf_ref[...]
            mx = jnp.max(logits, axis=1, keepdims=True)
            lse = mx + jnp.log(jnp.sum(jnp.exp(logits - mx), axis=1,
                                       keepdims=True))
            out_ref[...] = logits - lse

    full = lambda shp: pl.BlockSpec(shp, lambda i: (0,) * len(shp))
    return pl.pallas_call(
        body,
        grid=(nb,),
        in_specs=[
            pl.BlockSpec((NC, RB, w), lambda i: (0, i, 0)),
            pl.BlockSpec((RB, d), lambda i: (i, 0)),
            pl.BlockSpec((RB, d), lambda i: (i, 0)),
            full((1, d)), full((1, d)),
            pl.BlockSpec((RB, 1), lambda i: (i, 0)),
            full((ncls, d)), full((1, ncls)),
        ],
        out_specs=pl.BlockSpec((num_graphs, ncls), lambda i: (0, 0)),
        out_shape=jax.ShapeDtypeStruct((num_graphs, ncls), _f32),
        scratch_shapes=[
            pltpu.VMEM((num_graphs, d), _f32),
            pltpu.VMEM((num_graphs, d), _f32),
        ],
    )(acc2, xl2, xr2, att_row, bias_row, batchf, wf, bf)


def kernel(x, edge_index, edge_attr, batch, We, be, Wl1, bl1, Wr1, br1, att1,
           bias1, Wl2, bl2, Wr2, br2, att2, bias2, Wf, bf):
    n, f_in = x.shape
    src = edge_index[0]
    dst = edge_index[1]
    hid = We.shape[0]
    num_graphs = 64

    src2d = src.reshape(-1, EB)
    dst2d = dst.reshape(-1, EB)
    acc0 = _sc_scalar_scatter(dst2d, edge_attr.reshape(-1, EB), n)
    xl1, xr1 = _tc_pre(
        x, acc0,
        We[:, 0].reshape(1, hid), be.reshape(1, hid),
        Wl1[:, :f_in], Wl1[:, f_in:], bl1.reshape(1, -1),
        Wr1[:, :f_in], Wr1[:, f_in:], br1.reshape(1, -1),
    )
    att1f = att1.reshape(-1)
    dh = att1f.shape[0] // 2
    acc1a = _sc_edge_pass(xl1[:, :dh], xr1[:, :dh], src2d, dst2d,
                          att1f[:dh], heads=2)
    acc1b = _sc_edge_pass(xl1[:, dh:], xr1[:, dh:], src2d, dst2d,
                          att1f[dh:], heads=2)
    xl2, xr2 = _tc_mid(
        acc1a, acc1b, xl1, xr1, att1.reshape(1, -1), bias1.reshape(1, -1),
        Wl2, bl2.reshape(1, -1), Wr2, br2.reshape(1, -1),
    )
    acc2 = _sc_edge_pass(xl2, xr2, src2d, dst2d, att2.reshape(-1), heads=1)
    return _tc_post(
        acc2, xl2, xr2, att2.reshape(1, -1), bias2.reshape(1, -1),
        batch.astype(_f32).reshape(-1, 1), Wf, bf.reshape(1, -1), num_graphs,
    )


# speedup vs baseline: 6.8381x; 1.0064x over previous
import dataclasses
import functools

import jax
import jax.numpy as jnp
from jax import lax
from jax.experimental import pallas as pl
from jax.experimental.pallas import tpu as pltpu
from jax.experimental.pallas import tpu_sc as plsc

NC, NS, L = 2, 16, 16
NW = NC * NS
EB = 128
RB = 1000

_f32 = jnp.float32


def _vsc_mesh():
    return plsc.VectorSubcoreMesh(
        core_axis_name="c", subcore_axis_name="s", num_cores=NC, num_subcores=NS
    )


def _sc_params():
    cp = pltpu.CompilerParams()
    fields = pltpu.CompilerParams.__dataclass_fields__
    if "needs_layout_passes" in fields:
        cp = dataclasses.replace(cp, needs_layout_passes=False)
    if "use_tc_tiling_on_sc" in fields:
        cp = dataclasses.replace(cp, use_tc_tiling_on_sc=False)
    return cp


def _iota16():
    return lax.iota(jnp.int32, L)


def _zero_rows(ref, nrows, width):
    nfull = width // L
    tail = width - nfull * L

    @pl.loop(0, nrows)
    def _(r):
        for q in range(nfull):
            ref[r, pl.ds(q * L, L)] = jnp.zeros((L,), _f32)
        if tail:
            rid = jnp.full((L,), r, jnp.int32)
            colt = jnp.minimum(nfull * L + _iota16(), width - 1)
            plsc.store_scatter(ref, [rid, colt], jnp.zeros((L,), _f32),
                               mask=_iota16() < tail)


def _sc_scalar_scatter(dst, edge_attr, n_nodes):
    nblk = dst.shape[0]
    w0 = 8
    nb_main = nblk // NW
    tail = nblk - nb_main * NW
    rows_per = (n_nodes // NS) & ~7
    rem = n_nodes - rows_per * NS
    zr = 48

    @functools.partial(
        pl.kernel,
        out_type=jax.ShapeDtypeStruct((NC, n_nodes, w0), _f32),
        mesh=_vsc_mesh(),
        scratch_types=[
            pltpu.VMEM((nb_main + 1, EB), jnp.int32),
            pltpu.VMEM((nb_main + 1, EB), _f32),
            pltpu.VMEM((EB, w0), _f32),
            pltpu.VMEM((EB, w0), _f32),
            pltpu.VMEM((zr, w0), _f32),
            pltpu.VMEM_SHARED((n_nodes, w0), _f32),
            pltpu.SemaphoreType.DMA,
            pltpu.SemaphoreType.DMA,
        ],
        compiler_params=_sc_params(),
    )
    def k(dst_hbm, ea_hbm, out_hbm, dilo, ealo, rv0, rv1, zb, acc,
          ssem0, ssem1):
        cid = lax.axis_index("c")
        sid = lax.axis_index("s")
        wid = sid * NC + cid
        rvs = ((rv0, ssem0), (rv1, ssem1))

        _zero_rows(zb, zr, w0)
        _zero_rows(rv0, EB, w0)
        _zero_rows(rv1, EB, w0)

        for rv in (rv0, rv1):
            @pl.loop(0, EB // L)
            def _(g, rv=rv):
                rid = g * L + _iota16()
                cid1 = jnp.full((L,), 1, jnp.int32)
                plsc.store_scatter(rv, [rid, cid1], jnp.ones((L,), _f32))

        pltpu.sync_copy(dst_hbm.at[pl.ds(wid * nb_main, nb_main)],
                        dilo.at[pl.ds(0, nb_main)])
        pltpu.sync_copy(ea_hbm.at[pl.ds(wid * nb_main, nb_main)],
                        ealo.at[pl.ds(0, nb_main)])
        if tail:
            @pl.when(wid < tail)
            def _():
                pltpu.sync_copy(dst_hbm.at[NW * nb_main + wid],
                                dilo.at[nb_main])
                pltpu.sync_copy(ea_hbm.at[NW * nb_main + wid],
                                ealo.at[nb_main])

        @pl.loop(0, rows_per, step=zr)
        def _(j):
            pltpu.sync_copy(zb, acc.at[pl.ds(sid * rows_per + j, zr)])

        @pl.when(sid == NS - 1)
        def _():
            pltpu.sync_copy(zb.at[pl.ds(0, rem)],
                            acc.at[pl.ds(NS * rows_per, rem)])

        plsc.subcore_barrier()

        def fill(kb, p):
            rv, _ = rvs[p]

            @pl.loop(0, EB // L)
            def _(g):
                rid = g * L + _iota16()
                cid0 = jnp.zeros((L,), jnp.int32)
                plsc.store_scatter(rv, [rid, cid0],
                                   ealo[kb, pl.ds(g * L, L)])

        def issue_scatter(kb, p):
            rv, ssem_ = rvs[p]
            pltpu.async_copy(rv, acc.at[dilo.at[kb]], ssem_, add=True)

        def wait_scatter(kb, p):
            rv, ssem_ = rvs[p]
            pltpu.make_async_copy(rv, acc.at[dilo.at[kb]], ssem_).wait()

        @pl.loop(0, nb_main // 2)
        def _(j):
            for p in (0, 1):
                kb = 2 * j + p

                @pl.when(kb >= 2)
                def _():
                    wait_scatter(kb - 2, p)

                fill(kb, p)
                issue_scatter(kb, p)

        wait_scatter(nb_main - 2, 0)
        wait_scatter(nb_main - 1, 1)

        if tail:
            @pl.when(wid < tail)
            def _():
                fill(nb_main, 0)
                pltpu.sync_copy(rv0, acc.at[dilo.at[nb_main]], add=True)

        plsc.subcore_barrier()
        pltpu.sync_copy(
            acc.at[pl.ds(sid * rows_per, rows_per)],
            out_hbm.at[cid, pl.ds(sid * rows_per, rows_per)],
        )

        @pl.when(sid == NS - 1)
        def _():
            pltpu.sync_copy(
                acc.at[pl.ds(NS * rows_per, rem)],
                out_hbm.at[cid, pl.ds(NS * rows_per, rem)],
            )

    return k(dst, edge_attr)


def _sc_edge_pass(xl, xr, src, dst, attf, heads):
    n_nodes, d = xl.shape
    h_ = heads
    c_ = d // h_
    w = d + h_
    nblk = src.shape[0]
    nb_main = nblk // NW
    tail = nblk - nb_main * NW
    rows_per = (n_nodes // NS) & ~7
    rem = n_nodes - rows_per * NS
    zr = 48

    @functools.partial(
        pl.kernel,
        out_type=jax.ShapeDtypeStruct((NC, n_nodes, w), _f32),
        mesh=_vsc_mesh(),
        scratch_types=[
            pltpu.VMEM((nb_main + 1, EB), jnp.int32),
            pltpu.VMEM((nb_main + 1, EB), jnp.int32),
            pltpu.VMEM((EB, d), _f32),
            pltpu.VMEM((EB, d), _f32),
            pltpu.VMEM((EB, d), _f32),
            pltpu.VMEM((EB, d), _f32),
            pltpu.VMEM((EB, w), _f32),
            pltpu.VMEM((EB, w), _f32),
            pltpu.VMEM((d,), _f32),
            pltpu.VMEM((zr, w), _f32),
            pltpu.VMEM_SHARED((n_nodes, w), _f32),
            pltpu.SemaphoreType.DMA,
            pltpu.SemaphoreType.DMA,
            pltpu.SemaphoreType.DMA,
            pltpu.SemaphoreType.DMA,
        ],
        compiler_params=_sc_params(),
    )
    def k(xl_hbm, xr_hbm, src_hbm, dst_hbm, att_hbm, out_hbm,
          silo, dilo, xlb0, xlb1, xrb0, xrb1, ob0, ob1, att_v, zb, acc,
          gsem0, gsem1, ssem0, ssem1):
        cid = lax.axis_index("c")
        sid = lax.axis_index("s")
        wid = sid * NC + cid
        bufs = ((xlb0, xrb0, ob0, gsem0, ssem0),
                (xlb1, xrb1, ob1, gsem1, ssem1))

        pltpu.sync_copy(att_hbm, att_v)
        _zero_rows(zb, zr, w)

        pltpu.sync_copy(src_hbm.at[pl.ds(wid * nb_main, nb_main)],
                        silo.at[pl.ds(0, nb_main)])
        pltpu.sync_copy(dst_hbm.at[pl.ds(wid * nb_main, nb_main)],
                        dilo.at[pl.ds(0, nb_main)])
        if tail:
            @pl.when(wid < tail)
            def _():
                pltpu.sync_copy(src_hbm.at[NW * nb_main + wid],
                                silo.at[nb_main])
                pltpu.sync_copy(dst_hbm.at[NW * nb_main + wid],
                                dilo.at[nb_main])

        @pl.loop(0, rows_per, step=zr)
        def _(j):
            pltpu.sync_copy(zb, acc.at[pl.ds(sid * rows_per + j, zr)])

        @pl.when(sid == NS - 1)
        def _():
            pltpu.sync_copy(zb.at[pl.ds(0, rem)],
                            acc.at[pl.ds(NS * rows_per, rem)])

        plsc.subcore_barrier()

        def issue_gather(kb, p):
            xlb_, xrb_, _, gsem_, _ = bufs[p]
            pltpu.async_copy(xl_hbm.at[silo.at[kb]], xlb_, gsem_)
            pltpu.async_copy(xr_hbm.at[dilo.at[kb]], xrb_, gsem_)

        def wait_gather(kb, p):
            xlb_, xrb_, _, gsem_, _ = bufs[p]
            pltpu.make_async_copy(xl_hbm.at[silo.at[kb]], xlb_, gsem_).wait()
            pltpu.make_async_copy(xr_hbm.at[dilo.at[kb]], xrb_, gsem_).wait()

        def issue_scatter(kb, p):
            _, _, ob_, _, ssem_ = bufs[p]
            pltpu.async_copy(ob_, acc.at[dilo.at[kb]], ssem_, add=True)

        def wait_scatter(kb, p):
            _, _, ob_, _, ssem_ = bufs[p]
            pltpu.make_async_copy(ob_, acc.at[dilo.at[kb]], ssem_).wait()

        def compute(p):
            xlb_, xrb_, ob_, _, _ = bufs[p]
            attv = [att_v[pl.ds(j * L, L)] for j in range(d // L)]
            lane = _iota16()
            mask0 = lane == 0
            nv = c_ // L

            @plsc.parallel_loop(0, EB, unroll=4)
            def _(e):
                exs = []
                for h in range(h_):
                    t = None
                    xh = []
                    for u in range(nv):
                        j = h * nv + u
                        ds_ = pl.ds(j * L, L)
                        xlv = xlb_[e, ds_]
                        xh.append(xlv)
                        m = xlv + xrb_[e, ds_]
                        m = jnp.maximum(m, 0.2 * m)
                        tu = m * attv[j]
                        t = tu if t is None else t + tu
                    for s in (1, 2, 4, 8):
                        t = t + t.at[lane ^ s].get(mode="promise_in_bounds")
                    ex = jnp.exp(t)
                    exs.append(ex)
                    for u in range(nv):
                        ob_[e, pl.ds((h * nv + u) * L, L)] = xh[u] * ex
                dv = exs[0]
                for h in range(1, h_):
                    dv = jnp.where(lane == h, exs[h], dv)
                plsc.store_scatter(
                    ob_, [jnp.full((L,), e, jnp.int32),
                          jnp.full((L,), d, jnp.int32)
                          + jnp.minimum(lane, h_ - 1)],
                    dv, mask=lane < h_)

        issue_gather(0, 0)

        @pl.loop(0, nb_main // 2)
        def _(j):
            for p in (0, 1):
                kb = 2 * j + p
                if p == 0:
                    issue_gather(kb + 1, 1)
                else:
                    @pl.when(kb + 1 < nb_main)
                    def _():
                        issue_gather(kb + 1, 0)
                wait_gather(kb, p)

                @pl.when(kb >= 2)
                def _():
                    wait_scatter(kb - 2, p)

                compute(p)
                issue_scatter(kb, p)

        wait_scatter(nb_main - 2, 0)
        wait_scatter(nb_main - 1, 1)

        if tail:
            @pl.when(wid < tail)
            def _():
                issue_gather(nb_main, 0)
                wait_gather(nb_main, 0)
                compute(0)
                pltpu.sync_copy(ob0, acc.at[dilo.at[nb_main]], add=True)

        plsc.subcore_barrier()
        pltpu.sync_copy(
            acc.at[pl.ds(sid * rows_per, rows_per)],
            out_hbm.at[cid, pl.ds(sid * rows_per, rows_per)],
        )

        @pl.when(sid == NS - 1)
        def _():
            pltpu.sync_copy(
                acc.at[pl.ds(NS * rows_per, rem)],
                out_hbm.at[cid, pl.ds(NS * rows_per, rem)],
            )

    return k(xl, xr, src, dst, attf)


def _dg(a, b):
    return lax.dot_general(a, b, (((1,), (1,)), ((), ())),
                           preferred_element_type=_f32)


def _tc_pre(x, acc0, we_row, be_row, wlx, wlm, bl, wrx, wrm, br):
    n, f_in = x.shape
    nb = n // RB
    hid = we_row.shape[1]
    d_out = bl.shape[1]

    def body(x_ref, a_ref, we_ref, be_ref, wlx_ref, wlm_ref, bl_ref,
             wrx_ref, wrm_ref, br_ref, xl_ref, xr_ref):
        s = a_ref[0, :, 0:1] + a_ref[1, :, 0:1]
        c = a_ref[0, :, 1:2] + a_ref[1, :, 1:2]
        msg = s * we_ref[...] + c * be_ref[...]
        xv = x_ref[...]
        xl_ref[...] = _dg(xv, wlx_ref[...]) + _dg(msg, wlm_ref[...]) + bl_ref[...]
        xr_ref[...] = _dg(xv, wrx_ref[...]) + _dg(msg, wrm_ref[...]) + br_ref[...]

    full = lambda shp: pl.BlockSpec(shp, lambda i: (0,) * len(shp))
    return pl.pallas_call(
        body,
        grid=(nb,),
        in_specs=[
            pl.BlockSpec((RB, f_in), lambda i: (i, 0)),
            pl.BlockSpec((NC, RB, acc0.shape[2]), lambda i: (0, i, 0)),
            full((1, hid)), full((1, hid)),
            full((d_out, f_in)), full((d_out, hid)), full((1, d_out)),
            full((d_out, f_in)), full((d_out, hid)), full((1, d_out)),
        ],
        out_specs=[
            pl.BlockSpec((RB, d_out), lambda i: (i, 0)),
            pl.BlockSpec((RB, d_out), lambda i: (i, 0)),
        ],
        out_shape=[
            jax.ShapeDtypeStruct((n, d_out), _f32),
            jax.ShapeDtypeStruct((n, d_out), _f32),
        ],
    )(x, acc0, we_row, be_row, wlx, wlm, bl, wrx, wrm, br)


def _tc_mid(acc1a, acc1b, xl1, xr1, att_row, bias_row, wl2, bl2, wr2, br2):
    n, d = xl1.shape
    h_ = 4
    c_ = d // h_
    wh = d // 2 + 2
    nb = n // RB
    d2 = wl2.shape[0]

    def body(a0_ref, a1_ref, xl_ref, xr_ref, att_ref, bias_ref, wl2_ref,
             bl2_ref, wr2_ref, br2_ref, xl2_ref, xr2_ref):
        dh = d // 2
        num = jnp.concatenate(
            [a0_ref[0, :, 0:dh] + a0_ref[1, :, 0:dh],
             a1_ref[0, :, 0:dh] + a1_ref[1, :, 0:dh]], axis=1)
        den = jnp.concatenate(
            [a0_ref[0, :, dh:dh + 2] + a0_ref[1, :, dh:dh + 2],
             a1_ref[0, :, dh:dh + 2] + a1_ref[1, :, dh:dh + 2]], axis=1)
        xlv = xl_ref[...]
        m = xlv + xr_ref[...]
        m = jnp.maximum(m, 0.2 * m)
        t = m * att_ref[...]
        ci = lax.broadcasted_iota(jnp.int32, (d, h_), 0) // c_
        hi = lax.broadcasted_iota(jnp.int32, (d, h_), 1)
        sel = (ci == hi).astype(_f32)
        als = lax.dot_general(t, sel, (((1,), (0,)), ((), ())),
                              preferred_element_type=_f32)
        exs = jnp.exp(als)
        den = den + exs
        ex128 = _dg(exs, sel)
        den128 = _dg(den, sel)
        out = (num + ex128 * xlv) / (den128 + 1e-16) + bias_ref[...]
        h1 = jnp.where(out > 0, out, jnp.exp(out) - 1.0)
        xl2_ref[...] = _dg(h1, wl2_ref[...]) + bl2_ref[...]
        xr2_ref[...] = _dg(h1, wr2_ref[...]) + br2_ref[...]

    full = lambda shp: pl.BlockSpec(shp, lambda i: (0,) * len(shp))
    return pl.pallas_call(
        body,
        grid=(nb,),
        in_specs=[
            pl.BlockSpec((NC, RB, wh), lambda i: (0, i, 0)),
            pl.BlockSpec((NC, RB, wh), lambda i: (0, i, 0)),
            pl.BlockSpec((RB, d), lambda i: (i, 0)),
            pl.BlockSpec((RB, d), lambda i: (i, 0)),
            full((1, d)), full((1, d)),
            full((d2, d)), full((1, d2)),
            full((d2, d)), full((1, d2)),
        ],
        out_specs=[
            pl.BlockSpec((RB, d2), lambda i: (i, 0)),
            pl.BlockSpec((RB, d2), lambda i: (i, 0)),
        ],
        out_shape=[
            jax.ShapeDtypeStruct((n, d2), _f32),
            jax.ShapeDtypeStruct((n, d2), _f32),
        ],
    )(acc1a, acc1b, xl1, xr1, att_row, bias_row, wl2, bl2, wr2, br2)


def _tc_post(acc2, xl2, xr2, att_row, bias_row, batchf, wf, bf, num_graphs):
    n, d = xl2.shape
    w = d + 1
    nb = n // RB
    ncls = wf.shape[0]

    def body(a_ref, xl_ref, xr_ref, att_ref, bias_ref, b_ref, wf_ref, bf_ref,
             out_ref, sums_ref, cnts_ref):
        i = pl.program_id(0)

        @pl.when(i == 0)
        def _():
            sums_ref[...] = jnp.zeros_like(sums_ref)
            cnts_ref[...] = jnp.zeros_like(cnts_ref)

        num = a_ref[0, :, 0:d] + a_ref[1, :, 0:d]
        den = a_ref[0, :, d:d + 1] + a_ref[1, :, d:d + 1]
        xlv = xl_ref[...]
        m = xlv + xr_ref[...]
        m = jnp.maximum(m, 0.2 * m)
        t = m * att_ref[...]
        al = jnp.sum(t, axis=1, keepdims=True)
        exs = jnp.exp(al)
        out = (num + exs * xlv) / (den + exs + 1e-16) + bias_ref[...]
        h2 = jnp.where(out > 0, out, jnp.exp(out) - 1.0)
        gi = lax.broadcasted_iota(jnp.int32, (RB, num_graphs), 1).astype(_f32)
        on = (b_ref[...] == gi).astype(_f32)
        sums_ref[...] += lax.dot_general(on, h2, (((0,), (0,)), ((), ())),
                                         preferred_element_type=_f32)
        cnts_ref[...] += lax.dot_general(on, jnp.ones_like(h2),
                                         (((0,), (0,)), ((), ())),
                                         preferred_element_type=_f32)

        @pl.when(i == nb - 1)
        def _():
            pooled = sums_ref[...] / jnp.maximum(cnts_ref[...], 1.0)
            logits = _dg(pooled, wf_ref[...]) + bf_ref[...]
            mx = jnp.max(logits, axis=1, keepdims=True)
            lse = mx + jnp.log(jnp.sum(jnp.exp(logits - mx), axis=1,
                                       keepdims=True))
            out_ref[...] = logits - lse

    full = lambda shp: pl.BlockSpec(shp, lambda i: (0,) * len(shp))
    return pl.pallas_call(
        body,
        grid=(nb,),
        in_specs=[
            pl.BlockSpec((NC, RB, w), lambda i: (0, i, 0)),
            pl.BlockSpec((RB, d), lambda i: (i, 0)),
            pl.BlockSpec((RB, d), lambda i: (i, 0)),
            full((1, d)), full((1, d)),
            pl.BlockSpec((RB, 1), lambda i: (i, 0)),
            full((ncls, d)), full((1, ncls)),
        ],
        out_specs=pl.BlockSpec((num_graphs, ncls), lambda i: (0, 0)),
        out_shape=jax.ShapeDtypeStruct((num_graphs, ncls), _f32),
        scratch_shapes=[
            pltpu.VMEM((num_graphs, d), _f32),
            pltpu.VMEM((num_graphs, d), _f32),
        ],
    )(acc2, xl2, xr2, att_row, bias_row, batchf, wf, bf)


def kernel(x, edge_index, edge_attr, batch, We, be, Wl1, bl1, Wr1, br1, att1,
           bias1, Wl2, bl2, Wr2, br2, att2, bias2, Wf, bf):
    n, f_in = x.shape
    src = edge_index[0]
    dst = edge_index[1]
    hid = We.shape[0]
    num_graphs = 64

    src2d = src.reshape(-1, EB)
    dst2d = dst.reshape(-1, EB)
    acc0 = _sc_scalar_scatter(dst2d, edge_attr.reshape(-1, EB), n)
    xl1, xr1 = _tc_pre(
        x, acc0,
        We[:, 0].reshape(1, hid), be.reshape(1, hid),
        Wl1[:, :f_in], Wl1[:, f_in:], bl1.reshape(1, -1),
        Wr1[:, :f_in], Wr1[:, f_in:], br1.reshape(1, -1),
    )
    att1f = att1.reshape(-1)
    dh = att1f.shape[0] // 2
    acc1a = _sc_edge_pass(xl1[:, :dh], xr1[:, :dh], src2d, dst2d,
                          att1f[:dh], heads=2)
    acc1b = _sc_edge_pass(xl1[:, dh:], xr1[:, dh:], src2d, dst2d,
                          att1f[dh:], heads=2)
    xl2, xr2 = _tc_mid(
        acc1a, acc1b, xl1, xr1, att1.reshape(1, -1), bias1.reshape(1, -1),
        Wl2, bl2.reshape(1, -1), Wr2, br2.reshape(1, -1),
    )
    acc2 = _sc_edge_pass(xl2, xr2, src2d, dst2d, att2.reshape(-1), heads=1)
    return _tc_post(
        acc2, xl2, xr2, att2.reshape(1, -1), bias2.reshape(1, -1),
        batch.astype(_f32).reshape(-1, 1), Wf, bf.reshape(1, -1), num_graphs,
    )
